# baseline (speedup 1.0000x reference)
import functools

import jax
import jax.numpy as jnp
from jax import lax
from jax.experimental import pallas as pl
from jax.experimental.pallas import tpu as pltpu

B = 512
N = 100000
D = 64
H = 64
K = 96
NC = 100
CHUNK = 2048
NPAD = 100352
NSTEPS = NPAD // CHUNK
WIN = 128
NWIN = NPAD // WIN
NEG = -3.0e38


def _enc_body(x_ref, ew_ref, eb_ref, kw_ref, kb_ref, xe_ref, xk_ref):
    xe = jnp.dot(x_ref[...], ew_ref[...].T,
                 preferred_element_type=jnp.float32) + eb_ref[...]
    xe_ref[...] = xe
    xk_ref[...] = jnp.dot(xe, kw_ref[...].T,
                          preferred_element_type=jnp.float32) + kb_ref[...]


def _encode_queries(x, enc_w, enc_b, key_w, key_b):
    return pl.pallas_call(
        _enc_body,
        out_shape=(jax.ShapeDtypeStruct((B, H), jnp.float32),
                   jax.ShapeDtypeStruct((B, H), jnp.float32)),
    )(x, enc_w, enc_b.reshape(1, H), key_w, key_b.reshape(1, H))


def _scores_body(cx_ref, ew_ref, eb_ref, kw_ref, kb_ref, xk_ref,
                 ck_ref, sc_ref, wm_ref):
    i = pl.program_id(0)
    ce = jnp.dot(cx_ref[...], ew_ref[...].T,
                 preferred_element_type=jnp.float32) + eb_ref[...]
    ck = jnp.dot(ce, kw_ref[...].T,
                 preferred_element_type=jnp.float32) + kb_ref[...]
    ck_ref[...] = ck
    cn2 = jnp.sum(ck * ck, axis=1)
    xc = lax.dot_general(xk_ref[...], ck,
                         (((1,), (1,)), ((), ())),
                         preferred_element_type=jnp.float32)
    col = i * CHUNK + lax.broadcasted_iota(jnp.int32, (1, CHUNK), 1)
    sc = jnp.where(col < N, xc - 0.5 * cn2[None, :], NEG)
    sc_ref[...] = sc
    wm_ref[...] = jnp.max(sc.reshape(B, CHUNK // WIN, WIN), axis=2)[None]


def _stage_a(cx_pad, enc_w, enc_b, key_w, key_b, xk):
    return pl.pallas_call(
        _scores_body,
        grid=(NSTEPS,),
        in_specs=[
            pl.BlockSpec((CHUNK, D), lambda i: (i, 0)),
            pl.BlockSpec((H, D), lambda i: (0, 0)),
            pl.BlockSpec((1, H), lambda i: (0, 0)),
            pl.BlockSpec((H, H), lambda i: (0, 0)),
            pl.BlockSpec((1, H), lambda i: (0, 0)),
            pl.BlockSpec((B, H), lambda i: (0, 0)),
        ],
        out_specs=(
            pl.BlockSpec((CHUNK, H), lambda i: (i, 0)),
            pl.BlockSpec((B, CHUNK), lambda i: (0, i)),
            pl.BlockSpec((1, B, CHUNK // WIN), lambda i: (i, 0, 0)),
        ),
        out_shape=(
            jax.ShapeDtypeStruct((NPAD, H), jnp.float32),
            jax.ShapeDtypeStruct((B, NPAD), jnp.float32),
            jax.ShapeDtypeStruct((NSTEPS, B, CHUNK // WIN), jnp.float32),
        ),
    )(cx_pad, enc_w, enc_b.reshape(1, H), key_w, key_b.reshape(1, H), xk)


BBLK = 64
NCPAD = 128


def _final_body(xe_ref, xk_ref, ctxk_ref, lab_ref, le_ref,
                w1_ref, b1_ref, w2_ref, out_ref):
    xe = xe_ref[...]
    xk = xk_ref[...]
    ctxk = ctxk_ref[...]
    diff3 = xk[:, None, :] - ctxk
    d2 = jnp.sum(diff3 * diff3, axis=2)
    tv = -jnp.sqrt(jnp.maximum(d2, 1e-12))
    m = jnp.max(tv, axis=1, keepdims=True)
    e = jnp.exp(tv - m)
    attn = e / jnp.sum(e, axis=1, keepdims=True)

    diff = diff3.reshape(BBLK * K, H)
    h = jnp.dot(diff, w1_ref[...].T, preferred_element_type=jnp.float32)
    h = jnp.maximum(h + b1_ref[...], 0.0)
    h = jnp.dot(h, w2_ref[...].T, preferred_element_type=jnp.float32)

    iota_c = lax.broadcasted_iota(jnp.int32, (BBLK, K, NCPAD), 2)
    onehot = (lab_ref[...][:, :, None] == iota_c).astype(
        jnp.float32).reshape(BBLK * K, NCPAD)
    labv = jnp.dot(onehot, le_ref[...], preferred_element_type=jnp.float32)

    tot = (labv + h).reshape(BBLK, K, H)
    ctx = jnp.sum(attn[:, :, None] * tot, axis=1)
    out_ref[...] = xe + ctx


def _stage_c(xe, xk, ctxk, labels, label_emb_pad, t_w1, t_b1, t_w2):
    return pl.pallas_call(
        _final_body,
        grid=(B // BBLK,),
        in_specs=[
            pl.BlockSpec((BBLK, H), lambda i: (i, 0)),
            pl.BlockSpec((BBLK, H), lambda i: (i, 0)),
            pl.BlockSpec((BBLK, K, H), lambda i: (i, 0, 0)),
            pl.BlockSpec((BBLK, K), lambda i: (i, 0)),
            pl.BlockSpec((NCPAD, H), lambda i: (0, 0)),
            pl.BlockSpec((H, H), lambda i: (0, 0)),
            pl.BlockSpec((1, H), lambda i: (0, 0)),
            pl.BlockSpec((H, H), lambda i: (0, 0)),
        ],
        out_specs=pl.BlockSpec((BBLK, H), lambda i: (i, 0)),
        out_shape=jax.ShapeDtypeStruct((B, H), jnp.float32),
    )(xe, xk, ctxk, labels, label_emb_pad, t_w1, t_b1.reshape(1, H), t_w2)


def kernel(x, candidate_x, candidate_labels, enc_w, enc_b, key_w, key_b,
           val_w, val_b, label_emb, t_w1, t_b1, t_w2):
    del val_w, val_b
    candidate_labels = candidate_labels.astype(jnp.int32)
    cx_pad = jnp.pad(candidate_x, ((0, NPAD - N), (0, 0)))
    le_pad = jnp.pad(label_emb, ((0, NCPAD - NC), (0, 0)))

    xe, xk = _encode_queries(x, enc_w, enc_b, key_w, key_b)
    ck, scores, wmax = _stage_a(cx_pad, enc_w, enc_b, key_w, key_b, xk)

    del wmax
    _, top_idx = lax.top_k(scores[:, :N], K)
    ctxk = ck[top_idx]
    labels = candidate_labels[top_idx]

    return _stage_c(xe, xk, ctxk, labels, le_pad, t_w1, t_b1, t_w2)


# speedup vs baseline: 3.3271x; 3.3271x over previous
import functools

import jax
import jax.numpy as jnp
from jax import lax
from jax.experimental import pallas as pl
from jax.experimental.pallas import tpu as pltpu
from jax.experimental.pallas import tpu_sc as plsc

B = 512
N = 100000
D = 64
H = 64
K = 96
NC = 100
CHUNK = 2048
NPAD = 100352
NSTEPS = NPAD // CHUNK
WIN = 128
NWIN = NPAD // WIN
NEG = -3.0e38


def _enc_body(x_ref, ew_ref, eb_ref, kw_ref, kb_ref, xe_ref, xk_ref):
    xe = jnp.dot(x_ref[...], ew_ref[...].T,
                 preferred_element_type=jnp.float32) + eb_ref[...]
    xe_ref[...] = xe
    xk_ref[...] = jnp.dot(xe, kw_ref[...].T,
                          preferred_element_type=jnp.float32) + kb_ref[...]


def _encode_queries(x, enc_w, enc_b, key_w, key_b):
    return pl.pallas_call(
        _enc_body,
        out_shape=(jax.ShapeDtypeStruct((B, H), jnp.float32),
                   jax.ShapeDtypeStruct((B, H), jnp.float32)),
    )(x, enc_w, enc_b.reshape(1, H), key_w, key_b.reshape(1, H))


def _scores_body(cx_ref, lab_ref, ew_ref, eb_ref, kw_ref, kb_ref, xk_ref,
                 ck_ref, sc_ref, wm_ref):
    i = pl.program_id(0)
    ce = jnp.dot(cx_ref[...], ew_ref[...].T,
                 preferred_element_type=jnp.float32) + eb_ref[...]
    ck = jnp.dot(ce, kw_ref[...].T,
                 preferred_element_type=jnp.float32) + kb_ref[...]
    ck_ref[...] = jnp.concatenate(
        [ck, jnp.zeros((CHUNK, 63), jnp.float32), lab_ref[...]], axis=1)
    cn2 = jnp.sum(ck * ck, axis=1)
    xc = lax.dot_general(xk_ref[...], ck,
                         (((1,), (1,)), ((), ())),
                         preferred_element_type=jnp.float32)
    col = i * CHUNK + lax.broadcasted_iota(jnp.int32, (1, CHUNK), 1)
    sc = jnp.where(col < N, xc - 0.5 * cn2[None, :], NEG)
    sc_ref[...] = sc
    wm_ref[...] = jnp.max(sc.reshape(B, CHUNK // WIN, WIN), axis=2)[None]


def _stage_a(cx_pad, labf, enc_w, enc_b, key_w, key_b, xk):
    return pl.pallas_call(
        _scores_body,
        grid=(NSTEPS,),
        in_specs=[
            pl.BlockSpec((CHUNK, D), lambda i: (i, 0)),
            pl.BlockSpec((CHUNK, 1), lambda i: (i, 0)),
            pl.BlockSpec((H, D), lambda i: (0, 0)),
            pl.BlockSpec((1, H), lambda i: (0, 0)),
            pl.BlockSpec((H, H), lambda i: (0, 0)),
            pl.BlockSpec((1, H), lambda i: (0, 0)),
            pl.BlockSpec((B, H), lambda i: (0, 0)),
        ],
        out_specs=(
            pl.BlockSpec((CHUNK, 128), lambda i: (i, 0)),
            pl.BlockSpec((B, CHUNK), lambda i: (0, i)),
            pl.BlockSpec((1, B, CHUNK // WIN), lambda i: (i, 0, 0)),
        ),
        out_shape=(
            jax.ShapeDtypeStruct((NPAD, 128), jnp.float32),
            jax.ShapeDtypeStruct((B, NPAD), jnp.float32),
            jax.ShapeDtypeStruct((NSTEPS, B, CHUNK // WIN), jnp.float32),
        ),
    )(cx_pad, labf, enc_w, enc_b.reshape(1, H), key_w,
      key_b.reshape(1, H), xk)


CAP = 2048
NVROW = NPAD // 16
NVCAP = CAP // 16
NVWIN = NWIN // 16
ROWS_PER_W = B // 32

def _to_u32(f):
    ub = lax.bitcast_convert_type(f, jnp.uint32)
    neg = (ub >> jnp.uint32(31)) == jnp.uint32(1)
    return jnp.where(neg, ~ub, ub | jnp.uint32(0x80000000))


def _iota16():
    return lax.broadcasted_iota(jnp.int32, (16,), 0)


def _select_kth(read_u, nv, k):
    prefix = jnp.uint32(0)
    k_rem = jnp.int32(k)
    ones = jnp.ones((16,), jnp.int32)

    def hist_round(shift, prefix, k_rem, first, hist_ref):
        hist_ref[...] = jnp.zeros((16,), jnp.int32)
        sh = jnp.uint32(shift)

        def body(i, carry):
            u = read_u(i)
            if first:
                m = jnp.ones((16,), jnp.bool_)
            else:
                m = (u >> jnp.uint32(shift + 4)) == (
                    prefix >> jnp.uint32(shift + 4))
            digit = ((u >> sh) & jnp.uint32(15)).astype(jnp.int32)
            plsc.addupdate_scatter(hist_ref, [digit], ones, mask=m)
            return carry

        lax.fori_loop(0, nv, body, jnp.int32(0))
        h = hist_ref[...]
        rh = lax.rev(h, (0,))
        c = plsc.cumsum(rh)
        ge = c >= k_rem
        i_star = jnp.max(plsc.all_reduce_ffs(ge))
        cnt_gt = jnp.sum(jnp.where(_iota16() < i_star, rh, 0))
        d = (jnp.int32(15) - i_star).astype(jnp.uint32)
        prefix = prefix | (d << sh)
        k_rem = k_rem - cnt_gt
        return prefix, k_rem

    def run(hist_ref):
        p, kr = prefix, k_rem
        for r in range(8):
            p, kr = hist_round(28 - 4 * r, p, kr, r == 0, hist_ref)
        return p, kr

    return run


def _sc_body(scores, wmax, ck, ctxk_out,
             row_v, wmf_v, wmu_v, hist_v, svalf_v, svalu_v, sidx_v,
             fidx_v, ckrows_v, sem):
    wid = lax.axis_index("s") * 2 + lax.axis_index("c")
    neg = jnp.full((16,), NEG, jnp.float32)

    def do_row(j, carry):
        row = wid * ROWS_PER_W + j
        pltpu.sync_copy(scores.at[row], row_v)
        pltpu.sync_copy(wmax.at[row], wmf_v)

        def map_wm(i, c):
            wmu_v[pl.ds(i * 16, 16)] = _to_u32(wmf_v[pl.ds(i * 16, 16)])
            return c
        lax.fori_loop(0, NVWIN, map_wm, jnp.int32(0))

        def read_wm(i):
            return wmu_v[pl.ds(i * 16, 16)]
        lb_u, _ = _select_kth(read_wm, NVWIN, K)(hist_v)
        lb_uv = jnp.full((16,), lb_u)
        lb_f = jnp.min(lax.bitcast_convert_type(
            jnp.where((lb_uv >> jnp.uint32(31)) == jnp.uint32(1),
                      lb_uv & jnp.uint32(0x7FFFFFFF),
                      ~lb_uv),
            jnp.float32))

        def clr(i, c):
            svalf_v[pl.ds(i * 16, 16)] = neg
            return c
        lax.fori_loop(0, NVCAP, clr, jnp.int32(0))

        def collect(i, off):
            s = row_v[pl.ds(i * 16, 16)]
            m = s >= lb_f

            def take(off):
                mi = m.astype(jnp.int32)
                pc = plsc.cumsum(mi)
                pos = jnp.minimum(off + pc - 1, jnp.int32(CAP - 1))
                ivec = _iota16() + i * 16
                plsc.store_scatter(sidx_v, [pos], ivec, mask=m)
                plsc.store_scatter(svalf_v, [pos], s, mask=m)
                return off + plsc.all_reduce_population_count(m)

            return lax.cond(jnp.any(m), take, lambda o: o, off)

        off = lax.fori_loop(0, NVROW, collect,
                            jnp.zeros((16,), jnp.int32))
        n_surv = jnp.max(off)
        overflow = n_surv > jnp.int32(CAP)

        def map_sv(i, c):
            svalu_v[pl.ds(i * 16, 16)] = _to_u32(svalf_v[pl.ds(i * 16, 16)])
            return c
        lax.fori_loop(0, NVCAP, map_sv, jnp.int32(0))

        def read_sv(i):
            return svalu_v[pl.ds(i * 16, 16)]

        def read_row_u(i):
            return _to_u32(row_v[pl.ds(i * 16, 16)])

        v96_u, q_eq = lax.cond(
            overflow,
            lambda: _select_kth(read_row_u, NVROW, K)(hist_v),
            lambda: _select_kth(read_sv, NVCAP, K)(hist_v))
        v96_vec = jnp.full((16,), v96_u)

        def emit(read_u, read_idx, nv):
            def body(i, carry):
                nout, eq_seen = carry
                u = read_u(i)
                m_gt = u > v96_vec
                m_eq = u == v96_vec
                eqc = plsc.cumsum(m_eq.astype(jnp.int32))
                take_eq = m_eq & ((eq_seen + eqc) <= q_eq)
                m = m_gt | take_eq
                mi = m.astype(jnp.int32)
                pos = jnp.minimum(nout + plsc.cumsum(mi) - 1,
                                  jnp.int32(K - 1))
                plsc.store_scatter(fidx_v, [pos], read_idx(i), mask=m)
                nout = nout + plsc.all_reduce_population_count(m)
                eq_seen = eq_seen + plsc.all_reduce_population_count(m_eq)
                return nout, eq_seen

            return body

        zz = (jnp.zeros((16,), jnp.int32), jnp.zeros((16,), jnp.int32))

        def emit_surv(_):
            body = emit(read_sv, lambda i: sidx_v[pl.ds(i * 16, 16)], NVCAP)
            lax.fori_loop(0, NVCAP, body, zz)
            return jnp.int32(0)

        def emit_full(_):
            body = emit(read_row_u, lambda i: _iota16() + i * 16, NVROW)
            lax.fori_loop(0, NVROW, body, zz)
            return jnp.int32(0)

        lax.cond(overflow, emit_full, emit_surv, jnp.int32(0))

        pltpu.async_copy(ck.at[fidx_v], ckrows_v, sem).wait()
        pltpu.sync_copy(ckrows_v, ctxk_out.at[row])
        return carry

    lax.fori_loop(0, ROWS_PER_W, do_row, jnp.int32(0))


def _stage_b(scores, wmax, ck):
    mesh = plsc.VectorSubcoreMesh(core_axis_name="c", subcore_axis_name="s")
    f = pl.kernel(
        _sc_body,
        mesh=mesh,
        compiler_params=pltpu.CompilerParams(needs_layout_passes=False),
        out_type=jax.ShapeDtypeStruct((B, K, 128), jnp.float32),
        scratch_types=[
            pltpu.VMEM((NPAD,), jnp.float32),
            pltpu.VMEM((NWIN,), jnp.float32),
            pltpu.VMEM((NWIN,), jnp.uint32),
            pltpu.VMEM((16,), jnp.int32),
            pltpu.VMEM((CAP,), jnp.float32),
            pltpu.VMEM((CAP,), jnp.uint32),
            pltpu.VMEM((CAP,), jnp.int32),
            pltpu.VMEM((K,), jnp.int32),
            pltpu.VMEM((K, 128), jnp.float32),
            pltpu.SemaphoreType.DMA,
        ],
    )
    return f(scores, wmax, ck)


BBLK = 64
NCPAD = 128


def _final_body(xe_ref, xk_ref, ctxk_ref, le_ref,
                w1_ref, b1_ref, w2_ref, out_ref):
    xe = xe_ref[...]
    xk = xk_ref[...]
    ctxk = ctxk_ref[..., :H]
    labels = ctxk_ref[..., 127].astype(jnp.int32)
    diff3 = xk[:, None, :] - ctxk
    d2 = jnp.sum(diff3 * diff3, axis=2)
    tv = -jnp.sqrt(jnp.maximum(d2, 1e-12))
    m = jnp.max(tv, axis=1, keepdims=True)
    e = jnp.exp(tv - m)
    attn = e / jnp.sum(e, axis=1, keepdims=True)

    diff = diff3.reshape(BBLK * K, H)
    h = jnp.dot(diff, w1_ref[...].T, preferred_element_type=jnp.float32)
    h = jnp.maximum(h + b1_ref[...], 0.0)
    h = jnp.dot(h, w2_ref[...].T, preferred_element_type=jnp.float32)

    iota_c = lax.broadcasted_iota(jnp.int32, (BBLK, K, NCPAD), 2)
    onehot = (labels[:, :, None] == iota_c).astype(
        jnp.float32).reshape(BBLK * K, NCPAD)
    labv = jnp.dot(onehot, le_ref[...], preferred_element_type=jnp.float32)

    tot = (labv + h).reshape(BBLK, K, H)
    ctx = jnp.sum(attn[:, :, None] * tot, axis=1)
    out_ref[...] = xe + ctx


def _stage_c(xe, xk, ctxk, label_emb_pad, t_w1, t_b1, t_w2):
    return pl.pallas_call(
        _final_body,
        grid=(B // BBLK,),
        in_specs=[
            pl.BlockSpec((BBLK, H), lambda i: (i, 0)),
            pl.BlockSpec((BBLK, H), lambda i: (i, 0)),
            pl.BlockSpec((BBLK, K, 128), lambda i: (i, 0, 0)),
            pl.BlockSpec((NCPAD, H), lambda i: (0, 0)),
            pl.BlockSpec((H, H), lambda i: (0, 0)),
            pl.BlockSpec((1, H), lambda i: (0, 0)),
            pl.BlockSpec((H, H), lambda i: (0, 0)),
        ],
        out_specs=pl.BlockSpec((BBLK, H), lambda i: (i, 0)),
        out_shape=jax.ShapeDtypeStruct((B, H), jnp.float32),
    )(xe, xk, ctxk, label_emb_pad, t_w1, t_b1.reshape(1, H), t_w2)


def kernel(x, candidate_x, candidate_labels, enc_w, enc_b, key_w, key_b,
           val_w, val_b, label_emb, t_w1, t_b1, t_w2):
    del val_w, val_b
    labf = jnp.pad(candidate_labels.astype(jnp.float32), (0, NPAD - N))
    labf = labf.reshape(NPAD, 1)
    cx_pad = jnp.pad(candidate_x, ((0, NPAD - N), (0, 0)))
    le_pad = jnp.pad(label_emb, ((0, NCPAD - NC), (0, 0)))

    xe, xk = _encode_queries(x, enc_w, enc_b, key_w, key_b)
    ck, scores, wmax3 = _stage_a(cx_pad, labf, enc_w, enc_b, key_w,
                                 key_b, xk)
    wmax = jnp.transpose(wmax3, (1, 0, 2)).reshape(B, NWIN)

    ctxk = _stage_b(scores, wmax, ck)

    return _stage_c(xe, xk, ctxk, le_pad, t_w1, t_b1, t_w2)


# speedup vs baseline: 5.4512x; 1.6384x over previous
import functools

import jax
import jax.numpy as jnp
from jax import lax
from jax.experimental import pallas as pl
from jax.experimental.pallas import tpu as pltpu
from jax.experimental.pallas import tpu_sc as plsc

B = 512
N = 100000
D = 64
H = 64
K = 96
NC = 100
CHUNK = 2048
NPAD = 100352
NSTEPS = NPAD // CHUNK
WIN = 128
NWIN = NPAD // WIN
NEG = -3.0e38


def _enc_body(x_ref, ew_ref, eb_ref, kw_ref, kb_ref, xe_ref, xk_ref):
    xe = jnp.dot(x_ref[...], ew_ref[...].T,
                 preferred_element_type=jnp.float32) + eb_ref[...]
    xe_ref[...] = xe
    xk_ref[...] = jnp.dot(xe, kw_ref[...].T,
                          preferred_element_type=jnp.float32) + kb_ref[...]


def _encode_queries(x, enc_w, enc_b, key_w, key_b):
    return pl.pallas_call(
        _enc_body,
        out_shape=(jax.ShapeDtypeStruct((B, H), jnp.float32),
                   jax.ShapeDtypeStruct((B, H), jnp.float32)),
    )(x, enc_w, enc_b.reshape(1, H), key_w, key_b.reshape(1, H))


def _scores_body(cx_ref, lab_ref, ew_ref, eb_ref, kw_ref, kb_ref, xk_ref,
                 ck_ref, sc_ref, wm_ref):
    i = pl.program_id(0)
    ce = jnp.dot(cx_ref[...], ew_ref[...].T,
                 preferred_element_type=jnp.float32) + eb_ref[...]
    ck = jnp.dot(ce, kw_ref[...].T,
                 preferred_element_type=jnp.float32) + kb_ref[...]
    ck_ref[...] = jnp.concatenate(
        [ck, jnp.zeros((CHUNK, 63), jnp.float32), lab_ref[...]], axis=1)
    cn2 = jnp.sum(ck * ck, axis=1)
    xc = lax.dot_general(xk_ref[...], ck,
                         (((1,), (1,)), ((), ())),
                         preferred_element_type=jnp.float32)
    col = i * CHUNK + lax.broadcasted_iota(jnp.int32, (1, CHUNK), 1)
    sc = jnp.where(col < N, xc - 0.5 * cn2[None, :], NEG)
    sc_ref[...] = sc
    wm_ref[...] = jnp.max(sc.reshape(B, CHUNK // WIN, WIN), axis=2)[None]


def _stage_a(cx_pad, labf, enc_w, enc_b, key_w, key_b, xk):
    return pl.pallas_call(
        _scores_body,
        grid=(NSTEPS,),
        in_specs=[
            pl.BlockSpec((CHUNK, D), lambda i: (i, 0)),
            pl.BlockSpec((CHUNK, 1), lambda i: (i, 0)),
            pl.BlockSpec((H, D), lambda i: (0, 0)),
            pl.BlockSpec((1, H), lambda i: (0, 0)),
            pl.BlockSpec((H, H), lambda i: (0, 0)),
            pl.BlockSpec((1, H), lambda i: (0, 0)),
            pl.BlockSpec((B, H), lambda i: (0, 0)),
        ],
        out_specs=(
            pl.BlockSpec((CHUNK, 128), lambda i: (i, 0)),
            pl.BlockSpec((B, CHUNK), lambda i: (0, i)),
            pl.BlockSpec((1, B, CHUNK // WIN), lambda i: (i, 0, 0)),
        ),
        out_shape=(
            jax.ShapeDtypeStruct((NPAD, 128), jnp.float32),
            jax.ShapeDtypeStruct((B, NPAD), jnp.float32),
            jax.ShapeDtypeStruct((NSTEPS, B, CHUNK // WIN), jnp.float32),
        ),
    )(cx_pad, labf, enc_w, enc_b.reshape(1, H), key_w,
      key_b.reshape(1, H), xk)


CAP = 2048
NVROW = NPAD // 16
NVCAP = CAP // 16
NVWIN = NWIN // 16
ROWS_PER_W = B // 32

def _to_u32(f):
    ub = lax.bitcast_convert_type(f, jnp.uint32)
    neg = (ub >> jnp.uint32(31)) == jnp.uint32(1)
    return jnp.where(neg, ~ub, ub | jnp.uint32(0x80000000))


def _iota16():
    return lax.broadcasted_iota(jnp.int32, (16,), 0)


def _select_kth(read_u, nv, k):
    prefix = jnp.uint32(0)
    k_rem = jnp.int32(k)
    ones = jnp.ones((16,), jnp.int32)

    def hist_round(shift, prefix, k_rem, first, hist_ref):
        hist_ref[...] = jnp.zeros((16,), jnp.int32)
        sh = jnp.uint32(shift)

        def body(i, carry):
            u = read_u(i)
            if first:
                m = jnp.ones((16,), jnp.bool_)
            else:
                m = (u >> jnp.uint32(shift + 4)) == (
                    prefix >> jnp.uint32(shift + 4))
            digit = ((u >> sh) & jnp.uint32(15)).astype(jnp.int32)
            plsc.addupdate_scatter(hist_ref, [digit], ones, mask=m)
            return carry

        lax.fori_loop(0, nv, body, jnp.int32(0))
        h = hist_ref[...]
        rh = lax.rev(h, (0,))
        c = plsc.cumsum(rh)
        ge = c >= k_rem
        i_star = jnp.max(plsc.all_reduce_ffs(ge))
        cnt_gt = jnp.sum(jnp.where(_iota16() < i_star, rh, 0))
        d = (jnp.int32(15) - i_star).astype(jnp.uint32)
        prefix = prefix | (d << sh)
        k_rem = k_rem - cnt_gt
        return prefix, k_rem

    def run(hist_ref):
        p, kr = prefix, k_rem
        for r in range(8):
            p, kr = hist_round(28 - 4 * r, p, kr, r == 0, hist_ref)
        return p, kr

    return run


def _sc_body(scores, wmax, ck, ctxk_out,
             row_v, wmf_v, wmu_v, hist_v, svalf_v, svalu_v, sidx_v,
             fidx_v, ckrows_v, sem):
    wid = lax.axis_index("s") * 2 + lax.axis_index("c")
    neg = jnp.full((16,), NEG, jnp.float32)

    def do_row(j, carry):
        row = wid * ROWS_PER_W + j
        row_cp = pltpu.async_copy(scores.at[row], row_v, sem)
        pltpu.sync_copy(wmax.at[row], wmf_v)

        def map_wm(i, c):
            wmu_v[pl.ds(i * 16, 16)] = _to_u32(wmf_v[pl.ds(i * 16, 16)])
            return c
        lax.fori_loop(0, NVWIN, map_wm, jnp.int32(0))

        def read_wm(i):
            return wmu_v[pl.ds(i * 16, 16)]
        lb_u, _ = _select_kth(read_wm, NVWIN, K)(hist_v)
        lb_uv = jnp.full((16,), lb_u)
        lb_f = jnp.min(lax.bitcast_convert_type(
            jnp.where((lb_uv >> jnp.uint32(31)) == jnp.uint32(1),
                      lb_uv & jnp.uint32(0x7FFFFFFF),
                      ~lb_uv),
            jnp.float32))

        row_cp.wait()

        def grp_body(g, off):
            wmv = wmf_v[pl.ds(g * 16, 16)]

            def proc_grp(off):
                for t in range(16):
                    def proc(off, t=t):
                        w = g * 16 + t
                        for u in range(8):
                            s = row_v[pl.ds(w * 128 + u * 16, 16)]
                            m = s >= lb_f
                            pc = plsc.cumsum(m.astype(jnp.int32))
                            pos = jnp.minimum(off + pc - 1,
                                              jnp.int32(CAP - 1))
                            ivec = _iota16() + (w * 128 + u * 16)
                            plsc.store_scatter(sidx_v, [pos], ivec, mask=m)
                            plsc.store_scatter(svalf_v, [pos], s, mask=m)
                            off = off + plsc.all_reduce_population_count(m)
                        return off

                    off = lax.cond(wmv[t] >= lb_f, proc,
                                   lambda o: o, off)
                return off

            return lax.cond(jnp.any(wmv >= lb_f), proc_grp,
                            lambda o: o, off)

        off = lax.fori_loop(0, NWIN // 16, grp_body,
                            jnp.zeros((16,), jnp.int32))
        n_surv = jnp.max(off)
        overflow = n_surv > jnp.int32(CAP)
        nv_used = jnp.minimum((n_surv + 15) // 16, jnp.int32(NVCAP))

        def tail_clear(_):
            base = (n_surv // 16) * 16
            m_tail = _iota16() >= (n_surv - base)
            plsc.store_scatter(svalf_v, [base + _iota16()], neg,
                               mask=m_tail)
            return jnp.int32(0)
        lax.cond((n_surv % 16 != 0) & jnp.logical_not(overflow),
                 tail_clear, lambda _: jnp.int32(0), jnp.int32(0))

        def map_sv(i, c):
            svalu_v[pl.ds(i * 16, 16)] = _to_u32(svalf_v[pl.ds(i * 16, 16)])
            return c
        lax.fori_loop(0, nv_used, map_sv, jnp.int32(0))

        def read_sv(i):
            return svalu_v[pl.ds(i * 16, 16)]

        def read_row_u(i):
            return _to_u32(row_v[pl.ds(i * 16, 16)])

        v96_u, q_eq = lax.cond(
            overflow,
            lambda: _select_kth(read_row_u, NVROW, K)(hist_v),
            lambda: _select_kth(read_sv, nv_used, K)(hist_v))
        v96_vec = jnp.full((16,), v96_u)

        def emit(read_u, read_idx, nv):
            def body(i, carry):
                nout, eq_seen = carry
                u = read_u(i)
                m_gt = u > v96_vec
                m_eq = u == v96_vec
                eqc = plsc.cumsum(m_eq.astype(jnp.int32))
                take_eq = m_eq & ((eq_seen + eqc) <= q_eq)
                m = m_gt | take_eq
                mi = m.astype(jnp.int32)
                pos = jnp.minimum(nout + plsc.cumsum(mi) - 1,
                                  jnp.int32(K - 1))
                plsc.store_scatter(fidx_v, [pos], read_idx(i), mask=m)
                nout = nout + plsc.all_reduce_population_count(m)
                eq_seen = eq_seen + plsc.all_reduce_population_count(m_eq)
                return nout, eq_seen

            return body

        zz = (jnp.zeros((16,), jnp.int32), jnp.zeros((16,), jnp.int32))

        def emit_surv(_):
            body = emit(read_sv, lambda i: sidx_v[pl.ds(i * 16, 16)], NVCAP)
            lax.fori_loop(0, nv_used, body, zz)
            return jnp.int32(0)

        def emit_full(_):
            body = emit(read_row_u, lambda i: _iota16() + i * 16, NVROW)
            lax.fori_loop(0, NVROW, body, zz)
            return jnp.int32(0)

        lax.cond(overflow, emit_full, emit_surv, jnp.int32(0))

        pltpu.async_copy(ck.at[fidx_v], ckrows_v, sem).wait()
        pltpu.sync_copy(ckrows_v, ctxk_out.at[row])
        return carry

    lax.fori_loop(0, ROWS_PER_W, do_row, jnp.int32(0))


def _stage_b(scores, wmax, ck):
    mesh = plsc.VectorSubcoreMesh(core_axis_name="c", subcore_axis_name="s")
    f = pl.kernel(
        _sc_body,
        mesh=mesh,
        compiler_params=pltpu.CompilerParams(needs_layout_passes=False),
        out_type=jax.ShapeDtypeStruct((B, K, 128), jnp.float32),
        scratch_types=[
            pltpu.VMEM((NPAD,), jnp.float32),
            pltpu.VMEM((NWIN,), jnp.float32),
            pltpu.VMEM((NWIN,), jnp.uint32),
            pltpu.VMEM((16,), jnp.int32),
            pltpu.VMEM((CAP,), jnp.float32),
            pltpu.VMEM((CAP,), jnp.uint32),
            pltpu.VMEM((CAP,), jnp.int32),
            pltpu.VMEM((K,), jnp.int32),
            pltpu.VMEM((K, 128), jnp.float32),
            pltpu.SemaphoreType.DMA,
        ],
    )
    return f(scores, wmax, ck)


BBLK = 64
NCPAD = 128


def _final_body(xe_ref, xk_ref, ctxk_ref, le_ref,
                w1_ref, b1_ref, w2_ref, out_ref):
    xe = xe_ref[...]
    xk = xk_ref[...]
    ctxk = ctxk_ref[..., :H]
    labels = ctxk_ref[..., 127].astype(jnp.int32)
    diff3 = xk[:, None, :] - ctxk
    d2 = jnp.sum(diff3 * diff3, axis=2)
    tv = -jnp.sqrt(jnp.maximum(d2, 1e-12))
    m = jnp.max(tv, axis=1, keepdims=True)
    e = jnp.exp(tv - m)
    attn = e / jnp.sum(e, axis=1, keepdims=True)

    diff = diff3.reshape(BBLK * K, H)
    h = jnp.dot(diff, w1_ref[...].T, preferred_element_type=jnp.float32)
    h = jnp.maximum(h + b1_ref[...], 0.0)
    h = jnp.dot(h, w2_ref[...].T, preferred_element_type=jnp.float32)

    iota_c = lax.broadcasted_iota(jnp.int32, (BBLK, K, NCPAD), 2)
    onehot = (labels[:, :, None] == iota_c).astype(
        jnp.float32).reshape(BBLK * K, NCPAD)
    labv = jnp.dot(onehot, le_ref[...], preferred_element_type=jnp.float32)

    tot = (labv + h).reshape(BBLK, K, H)
    ctx = jnp.sum(attn[:, :, None] * tot, axis=1)
    out_ref[...] = xe + ctx


def _stage_c(xe, xk, ctxk, label_emb_pad, t_w1, t_b1, t_w2):
    return pl.pallas_call(
        _final_body,
        grid=(B // BBLK,),
        in_specs=[
            pl.BlockSpec((BBLK, H), lambda i: (i, 0)),
            pl.BlockSpec((BBLK, H), lambda i: (i, 0)),
            pl.BlockSpec((BBLK, K, 128), lambda i: (i, 0, 0)),
            pl.BlockSpec((NCPAD, H), lambda i: (0, 0)),
            pl.BlockSpec((H, H), lambda i: (0, 0)),
            pl.BlockSpec((1, H), lambda i: (0, 0)),
            pl.BlockSpec((H, H), lambda i: (0, 0)),
        ],
        out_specs=pl.BlockSpec((BBLK, H), lambda i: (i, 0)),
        out_shape=jax.ShapeDtypeStruct((B, H), jnp.float32),
    )(xe, xk, ctxk, label_emb_pad, t_w1, t_b1.reshape(1, H), t_w2)


def kernel(x, candidate_x, candidate_labels, enc_w, enc_b, key_w, key_b,
           val_w, val_b, label_emb, t_w1, t_b1, t_w2):
    del val_w, val_b
    labf = jnp.pad(candidate_labels.astype(jnp.float32), (0, NPAD - N))
    labf = labf.reshape(NPAD, 1)
    cx_pad = jnp.pad(candidate_x, ((0, NPAD - N), (0, 0)))
    le_pad = jnp.pad(label_emb, ((0, NCPAD - NC), (0, 0)))

    xe, xk = _encode_queries(x, enc_w, enc_b, key_w, key_b)
    ck, scores, wmax3 = _stage_a(cx_pad, labf, enc_w, enc_b, key_w,
                                 key_b, xk)
    wmax = jnp.transpose(wmax3, (1, 0, 2)).reshape(B, NWIN)

    ctxk = _stage_b(scores, wmax, ck)

    return _stage_c(xe, xk, ctxk, le_pad, t_w1, t_b1, t_w2)


# speedup vs baseline: 5.4570x; 1.0011x over previous
import functools

import jax
import jax.numpy as jnp
from jax import lax
from jax.experimental import pallas as pl
from jax.experimental.pallas import tpu as pltpu
from jax.experimental.pallas import tpu_sc as plsc

B = 512
N = 100000
D = 64
H = 64
K = 96
NC = 100
CHUNK = 2048
NPAD = 100352
NSTEPS = NPAD // CHUNK
WIN = 128
NWIN = NPAD // WIN
NEG = -3.0e38


def _enc_body(x_ref, ew_ref, eb_ref, kw_ref, kb_ref, xe_ref, xk_ref):
    xe = jnp.dot(x_ref[...], ew_ref[...].T,
                 preferred_element_type=jnp.float32) + eb_ref[...]
    xe_ref[...] = xe
    xk_ref[...] = jnp.dot(xe, kw_ref[...].T,
                          preferred_element_type=jnp.float32) + kb_ref[...]


def _encode_queries(x, enc_w, enc_b, key_w, key_b):
    return pl.pallas_call(
        _enc_body,
        out_shape=(jax.ShapeDtypeStruct((B, H), jnp.float32),
                   jax.ShapeDtypeStruct((B, H), jnp.float32)),
    )(x, enc_w, enc_b.reshape(1, H), key_w, key_b.reshape(1, H))


def _scores_body(cx_ref, lab_ref, ew_ref, eb_ref, kw_ref, kb_ref, xk_ref,
                 ck_ref, sc_ref, wm_ref):
    i = pl.program_id(0)
    ce = jnp.dot(cx_ref[...], ew_ref[...].T,
                 preferred_element_type=jnp.float32) + eb_ref[...]
    ck = jnp.dot(ce, kw_ref[...].T,
                 preferred_element_type=jnp.float32) + kb_ref[...]
    ck_ref[...] = jnp.concatenate(
        [ck, jnp.zeros((CHUNK, 63), jnp.float32), lab_ref[...]], axis=1)
    cn2 = jnp.sum(ck * ck, axis=1)
    xc = lax.dot_general(xk_ref[...], ck,
                         (((1,), (1,)), ((), ())),
                         preferred_element_type=jnp.float32)
    col = i * CHUNK + lax.broadcasted_iota(jnp.int32, (1, CHUNK), 1)
    sc = jnp.where(col < N, xc - 0.5 * cn2[None, :], NEG)
    sc_ref[...] = sc
    wm_ref[...] = jnp.max(sc.reshape(B, CHUNK // WIN, WIN), axis=2)[None]


def _stage_a(cx_pad, labf, enc_w, enc_b, key_w, key_b, xk):
    return pl.pallas_call(
        _scores_body,
        grid=(NSTEPS,),
        in_specs=[
            pl.BlockSpec((CHUNK, D), lambda i: (i, 0)),
            pl.BlockSpec((CHUNK, 1), lambda i: (i, 0)),
            pl.BlockSpec((H, D), lambda i: (0, 0)),
            pl.BlockSpec((1, H), lambda i: (0, 0)),
            pl.BlockSpec((H, H), lambda i: (0, 0)),
            pl.BlockSpec((1, H), lambda i: (0, 0)),
            pl.BlockSpec((B, H), lambda i: (0, 0)),
        ],
        out_specs=(
            pl.BlockSpec((CHUNK, 128), lambda i: (i, 0)),
            pl.BlockSpec((B, CHUNK), lambda i: (0, i)),
            pl.BlockSpec((1, B, CHUNK // WIN), lambda i: (i, 0, 0)),
        ),
        out_shape=(
            jax.ShapeDtypeStruct((NPAD, 128), jnp.float32),
            jax.ShapeDtypeStruct((B, NPAD), jnp.float32),
            jax.ShapeDtypeStruct((NSTEPS, B, CHUNK // WIN), jnp.float32),
        ),
    )(cx_pad, labf, enc_w, enc_b.reshape(1, H), key_w,
      key_b.reshape(1, H), xk)


CAP = 2048
NVROW = NPAD // 16
NVCAP = CAP // 16
NVWIN = NWIN // 16
ROWS_PER_W = B // 32

def _to_u32(f):
    ub = lax.bitcast_convert_type(f, jnp.uint32)
    neg = (ub >> jnp.uint32(31)) == jnp.uint32(1)
    return jnp.where(neg, ~ub, ub | jnp.uint32(0x80000000))


def _iota16():
    return lax.broadcasted_iota(jnp.int32, (16,), 0)


def _select_kth(read_u, nv, k, rounds=8):
    prefix = jnp.uint32(0)
    k_rem = jnp.int32(k)
    ones = jnp.ones((16,), jnp.int32)

    def hist_round(shift, prefix, k_rem, first, hist_ref):
        hist_ref[...] = jnp.zeros((16,), jnp.int32)
        sh = jnp.uint32(shift)

        def body(i, carry):
            u = read_u(i)
            if first:
                m = jnp.ones((16,), jnp.bool_)
            else:
                m = (u >> jnp.uint32(shift + 4)) == (
                    prefix >> jnp.uint32(shift + 4))
            digit = ((u >> sh) & jnp.uint32(15)).astype(jnp.int32)
            plsc.addupdate_scatter(hist_ref, [digit], ones, mask=m)
            return carry

        lax.fori_loop(0, nv, body, jnp.int32(0))
        h = hist_ref[...]
        rh = lax.rev(h, (0,))
        c = plsc.cumsum(rh)
        ge = c >= k_rem
        i_star = jnp.max(plsc.all_reduce_ffs(ge))
        cnt_gt = jnp.sum(jnp.where(_iota16() < i_star, rh, 0))
        d = (jnp.int32(15) - i_star).astype(jnp.uint32)
        prefix = prefix | (d << sh)
        k_rem = k_rem - cnt_gt
        return prefix, k_rem

    def run(hist_ref):
        p, kr = prefix, k_rem
        for r in range(rounds):
            p, kr = hist_round(28 - 4 * r, p, kr, r == 0, hist_ref)
        return p, kr

    return run


def _sc_body(scores, wmax, ck, ctxk_out,
             row_v, wmf_v, wmu_v, hist_v, svalf_v, svalu_v, sidx_v,
             fidx_v, ckrows_v, sem):
    wid = lax.axis_index("s") * 2 + lax.axis_index("c")
    neg = jnp.full((16,), NEG, jnp.float32)

    def do_row(j, carry):
        row = wid * ROWS_PER_W + j
        row_cp = pltpu.async_copy(scores.at[row], row_v, sem)
        pltpu.sync_copy(wmax.at[row], wmf_v)

        def map_wm(i, c):
            wmu_v[pl.ds(i * 16, 16)] = _to_u32(wmf_v[pl.ds(i * 16, 16)])
            return c
        lax.fori_loop(0, NVWIN, map_wm, jnp.int32(0))

        def read_wm(i):
            return wmu_v[pl.ds(i * 16, 16)]
        lb_u, _ = _select_kth(read_wm, NVWIN, K, rounds=4)(hist_v)
        lb_uv = jnp.full((16,), lb_u)
        lb_f = jnp.min(lax.bitcast_convert_type(
            jnp.where((lb_uv >> jnp.uint32(31)) == jnp.uint32(1),
                      lb_uv & jnp.uint32(0x7FFFFFFF),
                      ~lb_uv),
            jnp.float32))

        row_cp.wait()

        def grp_body(g, off):
            wmv = wmf_v[pl.ds(g * 16, 16)]

            def proc_grp(off):
                for t in range(16):
                    def proc(off, t=t):
                        w = g * 16 + t
                        for u in range(8):
                            s = row_v[pl.ds(w * 128 + u * 16, 16)]
                            m = s >= lb_f
                            pc = plsc.cumsum(m.astype(jnp.int32))
                            pos = jnp.minimum(off + pc - 1,
                                              jnp.int32(CAP - 1))
                            ivec = _iota16() + (w * 128 + u * 16)
                            plsc.store_scatter(sidx_v, [pos], ivec, mask=m)
                            plsc.store_scatter(svalf_v, [pos], s, mask=m)
                            off = off + plsc.all_reduce_population_count(m)
                        return off

                    off = lax.cond(wmv[t] >= lb_f, proc,
                                   lambda o: o, off)
                return off

            return lax.cond(jnp.any(wmv >= lb_f), proc_grp,
                            lambda o: o, off)

        off = lax.fori_loop(0, NWIN // 16, grp_body,
                            jnp.zeros((16,), jnp.int32))
        n_surv = jnp.max(off)
        overflow = n_surv > jnp.int32(CAP)
        nv_used = jnp.minimum((n_surv + 15) // 16, jnp.int32(NVCAP))

        def tail_clear(_):
            base = (n_surv // 16) * 16
            m_tail = _iota16() >= (n_surv - base)
            plsc.store_scatter(svalf_v, [base + _iota16()], neg,
                               mask=m_tail)
            return jnp.int32(0)
        lax.cond((n_surv % 16 != 0) & jnp.logical_not(overflow),
                 tail_clear, lambda _: jnp.int32(0), jnp.int32(0))

        def map_sv(i, c):
            svalu_v[pl.ds(i * 16, 16)] = _to_u32(svalf_v[pl.ds(i * 16, 16)])
            return c
        lax.fori_loop(0, nv_used, map_sv, jnp.int32(0))

        def read_sv(i):
            return svalu_v[pl.ds(i * 16, 16)]

        def read_row_u(i):
            return _to_u32(row_v[pl.ds(i * 16, 16)])

        v96_u, q_eq = lax.cond(
            overflow,
            lambda: _select_kth(read_row_u, NVROW, K)(hist_v),
            lambda: _select_kth(read_sv, nv_used, K)(hist_v))
        v96_vec = jnp.full((16,), v96_u)

        def emit(read_u, read_idx, nv):
            def body(i, carry):
                nout, eq_seen = carry
                u = read_u(i)
                m_gt = u > v96_vec
                m_eq = u == v96_vec
                eqc = plsc.cumsum(m_eq.astype(jnp.int32))
                take_eq = m_eq & ((eq_seen + eqc) <= q_eq)
                m = m_gt | take_eq
                mi = m.astype(jnp.int32)
                pos = jnp.minimum(nout + plsc.cumsum(mi) - 1,
                                  jnp.int32(K - 1))
                plsc.store_scatter(fidx_v, [pos], read_idx(i), mask=m)
                nout = nout + plsc.all_reduce_population_count(m)
                eq_seen = eq_seen + plsc.all_reduce_population_count(m_eq)
                return nout, eq_seen

            return body

        zz = (jnp.zeros((16,), jnp.int32), jnp.zeros((16,), jnp.int32))

        def emit_surv(_):
            body = emit(read_sv, lambda i: sidx_v[pl.ds(i * 16, 16)], NVCAP)
            lax.fori_loop(0, nv_used, body, zz)
            return jnp.int32(0)

        def emit_full(_):
            body = emit(read_row_u, lambda i: _iota16() + i * 16, NVROW)
            lax.fori_loop(0, NVROW, body, zz)
            return jnp.int32(0)

        lax.cond(overflow, emit_full, emit_surv, jnp.int32(0))

        pltpu.async_copy(ck.at[fidx_v], ckrows_v, sem).wait()
        pltpu.sync_copy(ckrows_v, ctxk_out.at[row])
        return carry

    lax.fori_loop(0, ROWS_PER_W, do_row, jnp.int32(0))


def _stage_b(scores, wmax, ck):
    mesh = plsc.VectorSubcoreMesh(core_axis_name="c", subcore_axis_name="s")
    f = pl.kernel(
        _sc_body,
        mesh=mesh,
        compiler_params=pltpu.CompilerParams(needs_layout_passes=False),
        out_type=jax.ShapeDtypeStruct((B, K, 128), jnp.float32),
        scratch_types=[
            pltpu.VMEM((NPAD,), jnp.float32),
            pltpu.VMEM((NWIN,), jnp.float32),
            pltpu.VMEM((NWIN,), jnp.uint32),
            pltpu.VMEM((16,), jnp.int32),
            pltpu.VMEM((CAP,), jnp.float32),
            pltpu.VMEM((CAP,), jnp.uint32),
            pltpu.VMEM((CAP,), jnp.int32),
            pltpu.VMEM((K,), jnp.int32),
            pltpu.VMEM((K, 128), jnp.float32),
            pltpu.SemaphoreType.DMA,
        ],
    )
    return f(scores, wmax, ck)


BBLK = 64
NCPAD = 128


def _final_body(xe_ref, xk_ref, ctxk_ref, le_ref,
                w1_ref, b1_ref, w2_ref, out_ref):
    xe = xe_ref[...]
    xk = xk_ref[...]
    ctxk = ctxk_ref[..., :H]
    labels = ctxk_ref[..., 127].astype(jnp.int32)
    diff3 = xk[:, None, :] - ctxk
    d2 = jnp.sum(diff3 * diff3, axis=2)
    tv = -jnp.sqrt(jnp.maximum(d2, 1e-12))
    m = jnp.max(tv, axis=1, keepdims=True)
    e = jnp.exp(tv - m)
    attn = e / jnp.sum(e, axis=1, keepdims=True)

    diff = diff3.reshape(BBLK * K, H)
    h = jnp.dot(diff, w1_ref[...].T, preferred_element_type=jnp.float32)
    h = jnp.maximum(h + b1_ref[...], 0.0)
    h = jnp.dot(h, w2_ref[...].T, preferred_element_type=jnp.float32)

    iota_c = lax.broadcasted_iota(jnp.int32, (BBLK, K, NCPAD), 2)
    onehot = (labels[:, :, None] == iota_c).astype(
        jnp.float32).reshape(BBLK * K, NCPAD)
    labv = jnp.dot(onehot, le_ref[...], preferred_element_type=jnp.float32)

    tot = (labv + h).reshape(BBLK, K, H)
    ctx = jnp.sum(attn[:, :, None] * tot, axis=1)
    out_ref[...] = xe + ctx


def _stage_c(xe, xk, ctxk, label_emb_pad, t_w1, t_b1, t_w2):
    return pl.pallas_call(
        _final_body,
        grid=(B // BBLK,),
        in_specs=[
            pl.BlockSpec((BBLK, H), lambda i: (i, 0)),
            pl.BlockSpec((BBLK, H), lambda i: (i, 0)),
            pl.BlockSpec((BBLK, K, 128), lambda i: (i, 0, 0)),
            pl.BlockSpec((NCPAD, H), lambda i: (0, 0)),
            pl.BlockSpec((H, H), lambda i: (0, 0)),
            pl.BlockSpec((1, H), lambda i: (0, 0)),
            pl.BlockSpec((H, H), lambda i: (0, 0)),
        ],
        out_specs=pl.BlockSpec((BBLK, H), lambda i: (i, 0)),
        out_shape=jax.ShapeDtypeStruct((B, H), jnp.float32),
    )(xe, xk, ctxk, label_emb_pad, t_w1, t_b1.reshape(1, H), t_w2)


def kernel(x, candidate_x, candidate_labels, enc_w, enc_b, key_w, key_b,
           val_w, val_b, label_emb, t_w1, t_b1, t_w2):
    del val_w, val_b
    labf = jnp.pad(candidate_labels.astype(jnp.float32), (0, NPAD - N))
    labf = labf.reshape(NPAD, 1)
    cx_pad = jnp.pad(candidate_x, ((0, NPAD - N), (0, 0)))
    le_pad = jnp.pad(label_emb, ((0, NCPAD - NC), (0, 0)))

    xe, xk = _encode_queries(x, enc_w, enc_b, key_w, key_b)
    ck, scores, wmax3 = _stage_a(cx_pad, labf, enc_w, enc_b, key_w,
                                 key_b, xk)
    wmax = jnp.transpose(wmax3, (1, 0, 2)).reshape(B, NWIN)

    ctxk = _stage_b(scores, wmax, ck)

    return _stage_c(xe, xk, ctxk, le_pad, t_w1, t_b1, t_w2)


# speedup vs baseline: 5.5146x; 1.0105x over previous
import functools

import jax
import jax.numpy as jnp
from jax import lax
from jax.experimental import pallas as pl
from jax.experimental.pallas import tpu as pltpu
from jax.experimental.pallas import tpu_sc as plsc

B = 512
N = 100000
D = 64
H = 64
K = 96
NC = 100
CHUNK = 2048
NPAD = 100352
NSTEPS = NPAD // CHUNK
WIN = 128
NWIN = NPAD // WIN
NEG = -3.0e38


def _enc_body(x_ref, ew_ref, eb_ref, kw_ref, kb_ref, xe_ref, xk_ref):
    xe = jnp.dot(x_ref[...], ew_ref[...].T,
                 preferred_element_type=jnp.float32) + eb_ref[...]
    xe_ref[...] = xe
    xk_ref[...] = jnp.dot(xe, kw_ref[...].T,
                          preferred_element_type=jnp.float32) + kb_ref[...]


def _encode_queries(x, enc_w, enc_b, key_w, key_b):
    return pl.pallas_call(
        _enc_body,
        out_shape=(jax.ShapeDtypeStruct((B, H), jnp.float32),
                   jax.ShapeDtypeStruct((B, H), jnp.float32)),
    )(x, enc_w, enc_b.reshape(1, H), key_w, key_b.reshape(1, H))


def _scores_body(cx_ref, lab_ref, ew_ref, eb_ref, kw_ref, kb_ref, xk_ref,
                 ck_ref, sc_ref, wm_ref):
    i = pl.program_id(0)
    ce = jnp.dot(cx_ref[...], ew_ref[...].T,
                 preferred_element_type=jnp.float32) + eb_ref[...]
    ck = jnp.dot(ce, kw_ref[...].T,
                 preferred_element_type=jnp.float32) + kb_ref[...]
    ck_ref[...] = jnp.concatenate(
        [ck, jnp.zeros((CHUNK, 63), jnp.float32), lab_ref[...]], axis=1)
    cn2 = jnp.sum(ck * ck, axis=1)
    xc = lax.dot_general(xk_ref[...], ck,
                         (((1,), (1,)), ((), ())),
                         preferred_element_type=jnp.float32)
    col = i * CHUNK + lax.broadcasted_iota(jnp.int32, (1, CHUNK), 1)
    sc = jnp.where(col < N, xc - 0.5 * cn2[None, :], NEG)
    sc_ref[...] = sc
    wm_ref[...] = jnp.max(sc.reshape(B, CHUNK // WIN, WIN), axis=2)[None]


def _stage_a(cx_pad, labf, enc_w, enc_b, key_w, key_b, xk):
    return pl.pallas_call(
        _scores_body,
        grid=(NSTEPS,),
        in_specs=[
            pl.BlockSpec((CHUNK, D), lambda i: (i, 0)),
            pl.BlockSpec((CHUNK, 1), lambda i: (i, 0)),
            pl.BlockSpec((H, D), lambda i: (0, 0)),
            pl.BlockSpec((1, H), lambda i: (0, 0)),
            pl.BlockSpec((H, H), lambda i: (0, 0)),
            pl.BlockSpec((1, H), lambda i: (0, 0)),
            pl.BlockSpec((B, H), lambda i: (0, 0)),
        ],
        out_specs=(
            pl.BlockSpec((CHUNK, 128), lambda i: (i, 0)),
            pl.BlockSpec((B, CHUNK), lambda i: (0, i)),
            pl.BlockSpec((1, B, CHUNK // WIN), lambda i: (i, 0, 0)),
        ),
        out_shape=(
            jax.ShapeDtypeStruct((NPAD, 128), jnp.float32),
            jax.ShapeDtypeStruct((B, NPAD), jnp.float32),
            jax.ShapeDtypeStruct((NSTEPS, B, CHUNK // WIN), jnp.float32),
        ),
    )(cx_pad, labf, enc_w, enc_b.reshape(1, H), key_w,
      key_b.reshape(1, H), xk)


CAP = 2048
NVROW = NPAD // 16
NVCAP = CAP // 16
NVWIN = NWIN // 16
ROWS_PER_W = B // 32

def _to_u32(f):
    ub = lax.bitcast_convert_type(f, jnp.uint32)
    neg = (ub >> jnp.uint32(31)) == jnp.uint32(1)
    return jnp.where(neg, ~ub, ub | jnp.uint32(0x80000000))


def _iota16():
    return lax.broadcasted_iota(jnp.int32, (16,), 0)


def _select_kth(read_u, nv, k, rounds=8):
    prefix = jnp.uint32(0)
    k_rem = jnp.int32(k)
    ones = jnp.ones((16,), jnp.int32)

    def hist_round(shift, prefix, k_rem, first, hist_ref):
        hist_ref[...] = jnp.zeros((16,), jnp.int32)
        sh = jnp.uint32(shift)

        def body(i, carry):
            u = read_u(i)
            if first:
                m = jnp.ones((16,), jnp.bool_)
            else:
                m = (u >> jnp.uint32(shift + 4)) == (
                    prefix >> jnp.uint32(shift + 4))
            digit = ((u >> sh) & jnp.uint32(15)).astype(jnp.int32)
            plsc.addupdate_scatter(hist_ref, [digit], ones, mask=m)
            return carry

        lax.fori_loop(0, nv, body, jnp.int32(0))
        h = hist_ref[...]
        rh = lax.rev(h, (0,))
        c = plsc.cumsum(rh)
        ge = c >= k_rem
        i_star = jnp.max(plsc.all_reduce_ffs(ge))
        cnt_gt = jnp.sum(jnp.where(_iota16() < i_star, rh, 0))
        d = (jnp.int32(15) - i_star).astype(jnp.uint32)
        prefix = prefix | (d << sh)
        k_rem = k_rem - cnt_gt
        return prefix, k_rem

    def run(hist_ref):
        p, kr = prefix, k_rem
        for r in range(rounds):
            p, kr = hist_round(28 - 4 * r, p, kr, r == 0, hist_ref)
        return p, kr

    return run


_EXP = 1


def _sc_body(scores, wmax, ck, ctxk_out,
             row_v, wmf_v, wmu_v, hist_v, svalf_v, svalu_v, sidx_v,
             fidx_v, ckrows_v, sem):
    wid = lax.axis_index("s") * 2 + lax.axis_index("c")
    neg = jnp.full((16,), NEG, jnp.float32)

    def do_row(j, carry):
        row = wid * ROWS_PER_W + j
        row_cp = pltpu.async_copy(scores.at[row], row_v, sem)
        pltpu.sync_copy(wmax.at[row], wmf_v)

        def map_wm(i, c):
            wmu_v[pl.ds(i * 16, 16)] = _to_u32(wmf_v[pl.ds(i * 16, 16)])
            return c
        lax.fori_loop(0, NVWIN, map_wm, jnp.int32(0))

        def read_wm(i):
            return wmu_v[pl.ds(i * 16, 16)]
        lb_u, _ = _select_kth(read_wm, NVWIN, K, rounds=4)(hist_v)
        lb_uv = jnp.full((16,), lb_u)
        lb_f = jnp.min(lax.bitcast_convert_type(
            jnp.where((lb_uv >> jnp.uint32(31)) == jnp.uint32(1),
                      lb_uv & jnp.uint32(0x7FFFFFFF),
                      ~lb_uv),
            jnp.float32))

        row_cp.wait()
        if _EXP >= 3:
            return carry

        def grp_body(g, off):
            wmv = wmf_v[pl.ds(g * 16, 16)]

            def proc_grp(off):
                for t in range(16):
                    def proc(off, t=t):
                        w = g * 16 + t
                        for u in range(8):
                            s = row_v[pl.ds(w * 128 + u * 16, 16)]
                            m = s >= lb_f
                            pc = plsc.cumsum(m.astype(jnp.int32))
                            pos = jnp.minimum(off + pc - 1,
                                              jnp.int32(CAP - 1))
                            ivec = _iota16() + (w * 128 + u * 16)
                            plsc.store_scatter(sidx_v, [pos], ivec, mask=m)
                            plsc.store_scatter(svalf_v, [pos], s, mask=m)
                            off = off + plsc.all_reduce_population_count(m)
                        return off

                    off = lax.cond(wmv[t] >= lb_f, proc,
                                   lambda o: o, off)
                return off

            return lax.cond(jnp.any(wmv >= lb_f), proc_grp,
                            lambda o: o, off)

        off = lax.fori_loop(0, NWIN // 16, grp_body,
                            jnp.zeros((16,), jnp.int32))
        n_surv = jnp.max(off)
        overflow = n_surv > jnp.int32(CAP)
        nv_used = jnp.minimum((n_surv + 15) // 16, jnp.int32(NVCAP))

        def tail_clear(_):
            base = (n_surv // 16) * 16
            m_tail = _iota16() >= (n_surv - base)
            plsc.store_scatter(svalf_v, [base + _iota16()], neg,
                               mask=m_tail)
            return jnp.int32(0)
        lax.cond((n_surv % 16 != 0) & jnp.logical_not(overflow),
                 tail_clear, lambda _: jnp.int32(0), jnp.int32(0))

        if _EXP >= 2:
            return carry

        def map_sv(i, c):
            svalu_v[pl.ds(i * 16, 16)] = _to_u32(svalf_v[pl.ds(i * 16, 16)])
            return c
        lax.fori_loop(0, nv_used, map_sv, jnp.int32(0))

        def read_sv(i):
            return svalu_v[pl.ds(i * 16, 16)]

        def read_row_u(i):
            return _to_u32(row_v[pl.ds(i * 16, 16)])

        v96_u, q_eq = lax.cond(
            overflow,
            lambda: _select_kth(read_row_u, NVROW, K)(hist_v),
            lambda: _select_kth(read_sv, nv_used, K)(hist_v))
        v96_vec = jnp.full((16,), v96_u)

        def emit(read_u, read_idx, nv):
            def body(i, carry):
                nout, eq_seen = carry
                u = read_u(i)
                m_gt = u > v96_vec
                m_eq = u == v96_vec
                eqc = plsc.cumsum(m_eq.astype(jnp.int32))
                take_eq = m_eq & ((eq_seen + eqc) <= q_eq)
                m = m_gt | take_eq
                mi = m.astype(jnp.int32)
                pos = jnp.minimum(nout + plsc.cumsum(mi) - 1,
                                  jnp.int32(K - 1))
                plsc.store_scatter(fidx_v, [pos], read_idx(i), mask=m)
                nout = nout + plsc.all_reduce_population_count(m)
                eq_seen = eq_seen + plsc.all_reduce_population_count(m_eq)
                return nout, eq_seen

            return body

        zz = (jnp.zeros((16,), jnp.int32), jnp.zeros((16,), jnp.int32))

        def emit_surv(_):
            body = emit(read_sv, lambda i: sidx_v[pl.ds(i * 16, 16)], NVCAP)
            lax.fori_loop(0, nv_used, body, zz)
            return jnp.int32(0)

        def emit_full(_):
            body = emit(read_row_u, lambda i: _iota16() + i * 16, NVROW)
            lax.fori_loop(0, NVROW, body, zz)
            return jnp.int32(0)

        lax.cond(overflow, emit_full, emit_surv, jnp.int32(0))

        if _EXP < 1:
            pltpu.async_copy(ck.at[fidx_v], ckrows_v, sem).wait()
            pltpu.sync_copy(ckrows_v, ctxk_out.at[row])
        return carry

    lax.fori_loop(0, ROWS_PER_W, do_row, jnp.int32(0))


def _stage_b(scores, wmax, ck):
    mesh = plsc.VectorSubcoreMesh(core_axis_name="c", subcore_axis_name="s")
    f = pl.kernel(
        _sc_body,
        mesh=mesh,
        compiler_params=pltpu.CompilerParams(needs_layout_passes=False),
        out_type=jax.ShapeDtypeStruct((B, K, 128), jnp.float32),
        scratch_types=[
            pltpu.VMEM((NPAD,), jnp.float32),
            pltpu.VMEM((NWIN,), jnp.float32),
            pltpu.VMEM((NWIN,), jnp.uint32),
            pltpu.VMEM((16,), jnp.int32),
            pltpu.VMEM((CAP,), jnp.float32),
            pltpu.VMEM((CAP,), jnp.uint32),
            pltpu.VMEM((CAP,), jnp.int32),
            pltpu.VMEM((K,), jnp.int32),
            pltpu.VMEM((K, 128), jnp.float32),
            pltpu.SemaphoreType.DMA,
        ],
    )
    return f(scores, wmax, ck)


BBLK = 64
NCPAD = 128


def _final_body(xe_ref, xk_ref, ctxk_ref, le_ref,
                w1_ref, b1_ref, w2_ref, out_ref):
    xe = xe_ref[...]
    xk = xk_ref[...]
    ctxk = ctxk_ref[..., :H]
    labels = ctxk_ref[..., 127].astype(jnp.int32)
    diff3 = xk[:, None, :] - ctxk
    d2 = jnp.sum(diff3 * diff3, axis=2)
    tv = -jnp.sqrt(jnp.maximum(d2, 1e-12))
    m = jnp.max(tv, axis=1, keepdims=True)
    e = jnp.exp(tv - m)
    attn = e / jnp.sum(e, axis=1, keepdims=True)

    diff = diff3.reshape(BBLK * K, H)
    h = jnp.dot(diff, w1_ref[...].T, preferred_element_type=jnp.float32)
    h = jnp.maximum(h + b1_ref[...], 0.0)
    h = jnp.dot(h, w2_ref[...].T, preferred_element_type=jnp.float32)

    iota_c = lax.broadcasted_iota(jnp.int32, (BBLK, K, NCPAD), 2)
    onehot = (labels[:, :, None] == iota_c).astype(
        jnp.float32).reshape(BBLK * K, NCPAD)
    labv = jnp.dot(onehot, le_ref[...], preferred_element_type=jnp.float32)

    tot = (labv + h).reshape(BBLK, K, H)
    ctx = jnp.sum(attn[:, :, None] * tot, axis=1)
    out_ref[...] = xe + ctx


def _stage_c(xe, xk, ctxk, label_emb_pad, t_w1, t_b1, t_w2):
    return pl.pallas_call(
        _final_body,
        grid=(B // BBLK,),
        in_specs=[
            pl.BlockSpec((BBLK, H), lambda i: (i, 0)),
            pl.BlockSpec((BBLK, H), lambda i: (i, 0)),
            pl.BlockSpec((BBLK, K, 128), lambda i: (i, 0, 0)),
            pl.BlockSpec((NCPAD, H), lambda i: (0, 0)),
            pl.BlockSpec((H, H), lambda i: (0, 0)),
            pl.BlockSpec((1, H), lambda i: (0, 0)),
            pl.BlockSpec((H, H), lambda i: (0, 0)),
        ],
        out_specs=pl.BlockSpec((BBLK, H), lambda i: (i, 0)),
        out_shape=jax.ShapeDtypeStruct((B, H), jnp.float32),
    )(xe, xk, ctxk, label_emb_pad, t_w1, t_b1.reshape(1, H), t_w2)


def kernel(x, candidate_x, candidate_labels, enc_w, enc_b, key_w, key_b,
           val_w, val_b, label_emb, t_w1, t_b1, t_w2):
    del val_w, val_b
    labf = jnp.pad(candidate_labels.astype(jnp.float32), (0, NPAD - N))
    labf = labf.reshape(NPAD, 1)
    cx_pad = jnp.pad(candidate_x, ((0, NPAD - N), (0, 0)))
    le_pad = jnp.pad(label_emb, ((0, NCPAD - NC), (0, 0)))

    xe, xk = _encode_queries(x, enc_w, enc_b, key_w, key_b)
    ck, scores, wmax3 = _stage_a(cx_pad, labf, enc_w, enc_b, key_w,
                                 key_b, xk)
    wmax = jnp.transpose(wmax3, (1, 0, 2)).reshape(B, NWIN)

    ctxk = _stage_b(scores, wmax, ck)

    return _stage_c(xe, xk, ctxk, le_pad, t_w1, t_b1, t_w2)


# speedup vs baseline: 5.5988x; 1.0153x over previous
import functools

import jax
import jax.numpy as jnp
from jax import lax
from jax.experimental import pallas as pl
from jax.experimental.pallas import tpu as pltpu
from jax.experimental.pallas import tpu_sc as plsc

B = 512
N = 100000
D = 64
H = 64
K = 96
NC = 100
CHUNK = 2048
NPAD = 100352
NSTEPS = NPAD // CHUNK
WIN = 128
NWIN = NPAD // WIN
NEG = -3.0e38


def _enc_body(x_ref, ew_ref, eb_ref, kw_ref, kb_ref, xe_ref, xk_ref):
    xe = jnp.dot(x_ref[...], ew_ref[...].T,
                 preferred_element_type=jnp.float32) + eb_ref[...]
    xe_ref[...] = xe
    xk_ref[...] = jnp.dot(xe, kw_ref[...].T,
                          preferred_element_type=jnp.float32) + kb_ref[...]


def _encode_queries(x, enc_w, enc_b, key_w, key_b):
    return pl.pallas_call(
        _enc_body,
        out_shape=(jax.ShapeDtypeStruct((B, H), jnp.float32),
                   jax.ShapeDtypeStruct((B, H), jnp.float32)),
    )(x, enc_w, enc_b.reshape(1, H), key_w, key_b.reshape(1, H))


def _scores_body(cx_ref, lab_ref, ew_ref, eb_ref, kw_ref, kb_ref, xk_ref,
                 ck_ref, sc_ref, wm_ref):
    i = pl.program_id(0)
    ce = jnp.dot(cx_ref[...], ew_ref[...].T,
                 preferred_element_type=jnp.float32) + eb_ref[...]
    ck = jnp.dot(ce, kw_ref[...].T,
                 preferred_element_type=jnp.float32) + kb_ref[...]
    ck_ref[...] = jnp.concatenate(
        [ck, jnp.zeros((CHUNK, 63), jnp.float32), lab_ref[...]], axis=1)
    cn2 = jnp.sum(ck * ck, axis=1)
    xc = lax.dot_general(xk_ref[...], ck,
                         (((1,), (1,)), ((), ())),
                         preferred_element_type=jnp.float32)
    col = i * CHUNK + lax.broadcasted_iota(jnp.int32, (1, CHUNK), 1)
    sc = jnp.where(col < N, xc - 0.5 * cn2[None, :], NEG)
    sc_ref[...] = sc
    wm_ref[...] = jnp.max(sc.reshape(B, CHUNK // WIN, WIN), axis=2)[None]


def _stage_a(cx_pad, labf, enc_w, enc_b, key_w, key_b, xk):
    return pl.pallas_call(
        _scores_body,
        grid=(NSTEPS,),
        in_specs=[
            pl.BlockSpec((CHUNK, D), lambda i: (i, 0)),
            pl.BlockSpec((CHUNK, 1), lambda i: (i, 0)),
            pl.BlockSpec((H, D), lambda i: (0, 0)),
            pl.BlockSpec((1, H), lambda i: (0, 0)),
            pl.BlockSpec((H, H), lambda i: (0, 0)),
            pl.BlockSpec((1, H), lambda i: (0, 0)),
            pl.BlockSpec((B, H), lambda i: (0, 0)),
        ],
        out_specs=(
            pl.BlockSpec((CHUNK, 128), lambda i: (i, 0)),
            pl.BlockSpec((B, CHUNK), lambda i: (0, i)),
            pl.BlockSpec((1, B, CHUNK // WIN), lambda i: (i, 0, 0)),
        ),
        out_shape=(
            jax.ShapeDtypeStruct((NPAD, 128), jnp.float32),
            jax.ShapeDtypeStruct((B, NPAD), jnp.float32),
            jax.ShapeDtypeStruct((NSTEPS, B, CHUNK // WIN), jnp.float32),
        ),
    )(cx_pad, labf, enc_w, enc_b.reshape(1, H), key_w,
      key_b.reshape(1, H), xk)


CAP = 2048
NVROW = NPAD // 16
NVCAP = CAP // 16
NVWIN = NWIN // 16
ROWS_PER_W = B // 32

def _to_u32(f):
    ub = lax.bitcast_convert_type(f, jnp.uint32)
    neg = (ub >> jnp.uint32(31)) == jnp.uint32(1)
    return jnp.where(neg, ~ub, ub | jnp.uint32(0x80000000))


def _iota16():
    return lax.broadcasted_iota(jnp.int32, (16,), 0)


def _select_kth(read_u, nv, k, rounds=8):
    prefix = jnp.uint32(0)
    k_rem = jnp.int32(k)
    ones = jnp.ones((16,), jnp.int32)

    def hist_round(shift, prefix, k_rem, first, hist_ref):
        hist_ref[...] = jnp.zeros((16,), jnp.int32)
        sh = jnp.uint32(shift)

        def body(i, carry):
            u = read_u(i)
            if first:
                m = jnp.ones((16,), jnp.bool_)
            else:
                m = (u >> jnp.uint32(shift + 4)) == (
                    prefix >> jnp.uint32(shift + 4))
            digit = ((u >> sh) & jnp.uint32(15)).astype(jnp.int32)
            plsc.addupdate_scatter(hist_ref, [digit], ones, mask=m)
            return carry

        lax.fori_loop(0, nv, body, jnp.int32(0))
        h = hist_ref[...]
        rh = lax.rev(h, (0,))
        c = plsc.cumsum(rh)
        ge = c >= k_rem
        i_star = jnp.max(plsc.all_reduce_ffs(ge))
        cnt_gt = jnp.sum(jnp.where(_iota16() < i_star, rh, 0))
        d = (jnp.int32(15) - i_star).astype(jnp.uint32)
        prefix = prefix | (d << sh)
        k_rem = k_rem - cnt_gt
        return prefix, k_rem

    def run(hist_ref):
        p, kr = prefix, k_rem
        for r in range(rounds):
            p, kr = hist_round(28 - 4 * r, p, kr, r == 0, hist_ref)
        return p, kr

    return run


_EXP = 0


def _sc_body(scores, wmax, ck, ctxk_out,
             row_v, wmf_v, wmu_v, hist_v, svalf_v, svalu_v, sidx_v,
             fidx_v, ckrows_v, sem):
    wid = lax.axis_index("s") * 2 + lax.axis_index("c")
    neg = jnp.full((16,), NEG, jnp.float32)

    def do_row(j, carry):
        row = wid * ROWS_PER_W + j
        row_cp = pltpu.async_copy(scores.at[row], row_v, sem)
        pltpu.sync_copy(wmax.at[row], wmf_v)

        def map_wm(i, c):
            wmu_v[pl.ds(i * 16, 16)] = _to_u32(wmf_v[pl.ds(i * 16, 16)])
            return c
        lax.fori_loop(0, NVWIN, map_wm, jnp.int32(0))

        def read_wm(i):
            return wmu_v[pl.ds(i * 16, 16)]
        lb_u, _ = _select_kth(read_wm, NVWIN, K, rounds=4)(hist_v)
        lb_uv = jnp.full((16,), lb_u)
        lb_f = jnp.min(lax.bitcast_convert_type(
            jnp.where((lb_uv >> jnp.uint32(31)) == jnp.uint32(1),
                      lb_uv & jnp.uint32(0x7FFFFFFF),
                      ~lb_uv),
            jnp.float32))

        row_cp.wait()
        if _EXP >= 3:
            return carry

        def grp_body(g, off):
            wmv = wmf_v[pl.ds(g * 16, 16)]
            hit = wmv >= lb_f
            bits = jnp.sum(jnp.where(hit, jnp.int32(1) << _iota16(),
                                     jnp.int32(0)))

            def proc_grp(off):
                for t in range(16):
                    def proc(off, t=t):
                        w = g * 16 + t
                        for u in range(8):
                            s = row_v[pl.ds(w * 128 + u * 16, 16)]
                            m = s >= lb_f
                            pc = plsc.cumsum(m.astype(jnp.int32))
                            pos = jnp.minimum(off + pc - 1,
                                              jnp.int32(CAP - 1))
                            ivec = _iota16() + (w * 128 + u * 16)
                            plsc.store_scatter(sidx_v, [pos], ivec, mask=m)
                            plsc.store_scatter(svalf_v, [pos], s, mask=m)
                            off = off + plsc.all_reduce_population_count(m)
                        return off

                    off = lax.cond(
                        ((bits >> jnp.int32(t)) & jnp.int32(1))
                        != jnp.int32(0),
                        proc, lambda o: o, off)
                return off

            return lax.cond(bits != jnp.int32(0), proc_grp,
                            lambda o: o, off)

        off = lax.fori_loop(0, NWIN // 16, grp_body,
                            jnp.zeros((16,), jnp.int32))
        n_surv = jnp.max(off)
        overflow = n_surv > jnp.int32(CAP)
        nv_used = jnp.minimum((n_surv + 15) // 16, jnp.int32(NVCAP))

        def tail_clear(_):
            base = (n_surv // 16) * 16
            m_tail = _iota16() >= (n_surv - base)
            plsc.store_scatter(svalf_v, [base + _iota16()], neg,
                               mask=m_tail)
            return jnp.int32(0)
        lax.cond((n_surv % 16 != 0) & jnp.logical_not(overflow),
                 tail_clear, lambda _: jnp.int32(0), jnp.int32(0))

        if _EXP >= 2:
            return carry

        def map_sv(i, c):
            svalu_v[pl.ds(i * 16, 16)] = _to_u32(svalf_v[pl.ds(i * 16, 16)])
            return c
        lax.fori_loop(0, nv_used, map_sv, jnp.int32(0))

        def read_sv(i):
            return svalu_v[pl.ds(i * 16, 16)]

        def read_row_u(i):
            return _to_u32(row_v[pl.ds(i * 16, 16)])

        v96_u, q_eq = lax.cond(
            overflow,
            lambda: _select_kth(read_row_u, NVROW, K)(hist_v),
            lambda: _select_kth(read_sv, nv_used, K)(hist_v))
        v96_vec = jnp.full((16,), v96_u)

        def emit(read_u, read_idx, nv):
            def body(i, carry):
                nout, eq_seen = carry
                u = read_u(i)
                m_gt = u > v96_vec
                m_eq = u == v96_vec
                eqc = plsc.cumsum(m_eq.astype(jnp.int32))
                take_eq = m_eq & ((eq_seen + eqc) <= q_eq)
                m = m_gt | take_eq
                mi = m.astype(jnp.int32)
                pos = jnp.minimum(nout + plsc.cumsum(mi) - 1,
                                  jnp.int32(K - 1))
                plsc.store_scatter(fidx_v, [pos], read_idx(i), mask=m)
                nout = nout + plsc.all_reduce_population_count(m)
                eq_seen = eq_seen + plsc.all_reduce_population_count(m_eq)
                return nout, eq_seen

            return body

        zz = (jnp.zeros((16,), jnp.int32), jnp.zeros((16,), jnp.int32))

        def emit_surv(_):
            body = emit(read_sv, lambda i: sidx_v[pl.ds(i * 16, 16)], NVCAP)
            lax.fori_loop(0, nv_used, body, zz)
            return jnp.int32(0)

        def emit_full(_):
            body = emit(read_row_u, lambda i: _iota16() + i * 16, NVROW)
            lax.fori_loop(0, NVROW, body, zz)
            return jnp.int32(0)

        lax.cond(overflow, emit_full, emit_surv, jnp.int32(0))

        if _EXP < 1:
            pltpu.async_copy(ck.at[fidx_v], ckrows_v, sem).wait()
            pltpu.sync_copy(ckrows_v, ctxk_out.at[row])
        return carry

    lax.fori_loop(0, ROWS_PER_W, do_row, jnp.int32(0))


def _stage_b(scores, wmax, ck):
    mesh = plsc.VectorSubcoreMesh(core_axis_name="c", subcore_axis_name="s")
    f = pl.kernel(
        _sc_body,
        mesh=mesh,
        compiler_params=pltpu.CompilerParams(needs_layout_passes=False),
        out_type=jax.ShapeDtypeStruct((B, K, 128), jnp.float32),
        scratch_types=[
            pltpu.VMEM((NPAD,), jnp.float32),
            pltpu.VMEM((NWIN,), jnp.float32),
            pltpu.VMEM((NWIN,), jnp.uint32),
            pltpu.VMEM((16,), jnp.int32),
            pltpu.VMEM((CAP,), jnp.float32),
            pltpu.VMEM((CAP,), jnp.uint32),
            pltpu.VMEM((CAP,), jnp.int32),
            pltpu.VMEM((K,), jnp.int32),
            pltpu.VMEM((K, 128), jnp.float32),
            pltpu.SemaphoreType.DMA,
        ],
    )
    return f(scores, wmax, ck)


BBLK = 64
NCPAD = 128


def _final_body(xe_ref, xk_ref, ctxk_ref, le_ref,
                w1_ref, b1_ref, w2_ref, out_ref):
    xe = xe_ref[...]
    xk = xk_ref[...]
    ctxk = ctxk_ref[..., :H]
    labels = ctxk_ref[..., 127].astype(jnp.int32)
    diff3 = xk[:, None, :] - ctxk
    d2 = jnp.sum(diff3 * diff3, axis=2)
    tv = -jnp.sqrt(jnp.maximum(d2, 1e-12))
    m = jnp.max(tv, axis=1, keepdims=True)
    e = jnp.exp(tv - m)
    attn = e / jnp.sum(e, axis=1, keepdims=True)

    diff = diff3.reshape(BBLK * K, H)
    h = jnp.dot(diff, w1_ref[...].T, preferred_element_type=jnp.float32)
    h = jnp.maximum(h + b1_ref[...], 0.0)
    h = jnp.dot(h, w2_ref[...].T, preferred_element_type=jnp.float32)

    iota_c = lax.broadcasted_iota(jnp.int32, (BBLK, K, NCPAD), 2)
    onehot = (labels[:, :, None] == iota_c).astype(
        jnp.float32).reshape(BBLK * K, NCPAD)
    labv = jnp.dot(onehot, le_ref[...], preferred_element_type=jnp.float32)

    tot = (labv + h).reshape(BBLK, K, H)
    ctx = jnp.sum(attn[:, :, None] * tot, axis=1)
    out_ref[...] = xe + ctx


def _stage_c(xe, xk, ctxk, label_emb_pad, t_w1, t_b1, t_w2):
    return pl.pallas_call(
        _final_body,
        grid=(B // BBLK,),
        in_specs=[
            pl.BlockSpec((BBLK, H), lambda i: (i, 0)),
            pl.BlockSpec((BBLK, H), lambda i: (i, 0)),
            pl.BlockSpec((BBLK, K, 128), lambda i: (i, 0, 0)),
            pl.BlockSpec((NCPAD, H), lambda i: (0, 0)),
            pl.BlockSpec((H, H), lambda i: (0, 0)),
            pl.BlockSpec((1, H), lambda i: (0, 0)),
            pl.BlockSpec((H, H), lambda i: (0, 0)),
        ],
        out_specs=pl.BlockSpec((BBLK, H), lambda i: (i, 0)),
        out_shape=jax.ShapeDtypeStruct((B, H), jnp.float32),
    )(xe, xk, ctxk, label_emb_pad, t_w1, t_b1.reshape(1, H), t_w2)


def kernel(x, candidate_x, candidate_labels, enc_w, enc_b, key_w, key_b,
           val_w, val_b, label_emb, t_w1, t_b1, t_w2):
    del val_w, val_b
    labf = jnp.pad(candidate_labels.astype(jnp.float32), (0, NPAD - N))
    labf = labf.reshape(NPAD, 1)
    cx_pad = jnp.pad(candidate_x, ((0, NPAD - N), (0, 0)))
    le_pad = jnp.pad(label_emb, ((0, NCPAD - NC), (0, 0)))

    xe, xk = _encode_queries(x, enc_w, enc_b, key_w, key_b)
    ck, scores, wmax3 = _stage_a(cx_pad, labf, enc_w, enc_b, key_w,
                                 key_b, xk)
    wmax = jnp.transpose(wmax3, (1, 0, 2)).reshape(B, NWIN)

    ctxk = _stage_b(scores, wmax, ck)

    return _stage_c(xe, xk, ctxk, le_pad, t_w1, t_b1, t_w2)


# speedup vs baseline: 17.1976x; 3.0716x over previous
import functools

import jax
import jax.numpy as jnp
from jax import lax
from jax.experimental import pallas as pl
from jax.experimental.pallas import tpu as pltpu
from jax.experimental.pallas import tpu_sc as plsc

B = 512
N = 100000
D = 64
H = 64
K = 96
NC = 100
CHUNK = 2048
NPAD = 100352
NSTEPS = NPAD // CHUNK
WIN = 128
NWIN = NPAD // WIN
NEG = -3.0e38


def _enc_body(x_ref, ew_ref, eb_ref, kw_ref, kb_ref, xe_ref, xk_ref):
    xe = jnp.dot(x_ref[...], ew_ref[...].T,
                 preferred_element_type=jnp.float32) + eb_ref[...]
    xe_ref[...] = xe
    xk_ref[...] = jnp.dot(xe, kw_ref[...].T,
                          preferred_element_type=jnp.float32) + kb_ref[...]


def _encode_queries(x, enc_w, enc_b, key_w, key_b):
    return pl.pallas_call(
        _enc_body,
        out_shape=(jax.ShapeDtypeStruct((B, H), jnp.float32),
                   jax.ShapeDtypeStruct((B, H), jnp.float32)),
    )(x, enc_w, enc_b.reshape(1, H), key_w, key_b.reshape(1, H))


def _scores_body(cx_ref, lab_ref, ew_ref, eb_ref, kw_ref, kb_ref, xk_ref,
                 ck_ref, sc_ref, wm_ref):
    i = pl.program_id(0)
    ce = jnp.dot(cx_ref[...], ew_ref[...].T,
                 preferred_element_type=jnp.float32) + eb_ref[...]
    ck = jnp.dot(ce, kw_ref[...].T,
                 preferred_element_type=jnp.float32) + kb_ref[...]
    ck_ref[...] = jnp.concatenate(
        [ck, jnp.zeros((CHUNK, 63), jnp.float32), lab_ref[...]], axis=1)
    cn2 = jnp.sum(ck * ck, axis=1)
    xc = lax.dot_general(xk_ref[...], ck,
                         (((1,), (1,)), ((), ())),
                         preferred_element_type=jnp.float32)
    col = i * CHUNK + lax.broadcasted_iota(jnp.int32, (1, CHUNK), 1)
    sc = jnp.where(col < N, xc - 0.5 * cn2[None, :], NEG)
    sc_ref[...] = sc
    wm_ref[...] = jnp.max(sc.reshape(B, CHUNK // WIN, WIN), axis=2)[None]


def _stage_a(cx_pad, labf, enc_w, enc_b, key_w, key_b, xk):
    return pl.pallas_call(
        _scores_body,
        grid=(NSTEPS,),
        in_specs=[
            pl.BlockSpec((CHUNK, D), lambda i: (i, 0)),
            pl.BlockSpec((CHUNK, 1), lambda i: (i, 0)),
            pl.BlockSpec((H, D), lambda i: (0, 0)),
            pl.BlockSpec((1, H), lambda i: (0, 0)),
            pl.BlockSpec((H, H), lambda i: (0, 0)),
            pl.BlockSpec((1, H), lambda i: (0, 0)),
            pl.BlockSpec((B, H), lambda i: (0, 0)),
        ],
        out_specs=(
            pl.BlockSpec((CHUNK, 128), lambda i: (i, 0)),
            pl.BlockSpec((B, CHUNK), lambda i: (0, i)),
            pl.BlockSpec((1, B, CHUNK // WIN), lambda i: (i, 0, 0)),
        ),
        out_shape=(
            jax.ShapeDtypeStruct((NPAD, 128), jnp.float32),
            jax.ShapeDtypeStruct((B, NPAD), jnp.float32),
            jax.ShapeDtypeStruct((NSTEPS, B, CHUNK // WIN), jnp.float32),
        ),
    )(cx_pad, labf, enc_w, enc_b.reshape(1, H), key_w,
      key_b.reshape(1, H), xk)


CAP = 4096
NVROW = NPAD // 16
NVCAP = CAP // 16
NVWIN = NWIN // 16
ROWS_PER_W = B // 32

def _to_u32(f):
    ub = lax.bitcast_convert_type(f, jnp.uint32)
    neg = (ub >> jnp.uint32(31)) == jnp.uint32(1)
    return jnp.where(neg, ~ub, ub | jnp.uint32(0x80000000))


def _iota16():
    return lax.broadcasted_iota(jnp.int32, (16,), 0)


def _select_kth(read_u, nv, k, rounds=8):
    prefix = jnp.uint32(0)
    k_rem = jnp.int32(k)
    ones = jnp.ones((16,), jnp.int32)

    def hist_round(shift, prefix, k_rem, first, hist_ref):
        hist_ref[...] = jnp.zeros((16,), jnp.int32)
        sh = jnp.uint32(shift)

        def body(i, carry):
            u = read_u(i)
            if first:
                m = jnp.ones((16,), jnp.bool_)
            else:
                m = (u >> jnp.uint32(shift + 4)) == (
                    prefix >> jnp.uint32(shift + 4))
            digit = ((u >> sh) & jnp.uint32(15)).astype(jnp.int32)
            plsc.addupdate_scatter(hist_ref, [digit], ones, mask=m)
            return carry

        lax.fori_loop(0, nv, body, jnp.int32(0))
        h = hist_ref[...]
        rh = lax.rev(h, (0,))
        c = plsc.cumsum(rh)
        ge = c >= k_rem
        i_star = jnp.max(plsc.all_reduce_ffs(ge))
        cnt_gt = jnp.sum(jnp.where(_iota16() < i_star, rh, 0))
        d = (jnp.int32(15) - i_star).astype(jnp.uint32)
        prefix = prefix | (d << sh)
        k_rem = k_rem - cnt_gt
        return prefix, k_rem

    def run(hist_ref):
        p, kr = prefix, k_rem
        for r in range(rounds):
            p, kr = hist_round(28 - 4 * r, p, kr, r == 0, hist_ref)
        return p, kr

    return run


_EXP = 0


def _sc_body(scores, wmax, ck, ctxk_out,
             row_v, wmf_v, wmu_v, hist_v, svalf_v, svalu_v, sidx_v,
             fidx_v, ckrows_v, sem):
    wid = lax.axis_index("s") * 2 + lax.axis_index("c")
    neg = jnp.full((16,), NEG, jnp.float32)

    def do_row(j, carry):
        row = wid * ROWS_PER_W + j
        row_cp = pltpu.async_copy(scores.at[row], row_v, sem)
        pltpu.sync_copy(wmax.at[row], wmf_v)

        def map_wm(i, c):
            wmu_v[pl.ds(i * 16, 16)] = _to_u32(wmf_v[pl.ds(i * 16, 16)])
            return c
        lax.fori_loop(0, NVWIN, map_wm, jnp.int32(0))

        def read_wm(i):
            return wmu_v[pl.ds(i * 16, 16)]
        lb_u, _ = _select_kth(read_wm, NVWIN, K, rounds=4)(hist_v)
        lb_uv = jnp.full((16,), lb_u)
        lb_f = jnp.min(lax.bitcast_convert_type(
            jnp.where((lb_uv >> jnp.uint32(31)) == jnp.uint32(1),
                      lb_uv & jnp.uint32(0x7FFFFFFF),
                      ~lb_uv),
            jnp.float32))

        row_cp.wait()
        if _EXP >= 3:
            return carry

        def grp_body(g, off):
            wmv = wmf_v[pl.ds(g * 16, 16)]
            hit = wmv >= lb_f
            bits = jnp.sum(jnp.where(hit, jnp.int32(1) << _iota16(),
                                     jnp.int32(0)))
            if _EXP == 5:
                bits = bits & jnp.int32(0)

            def proc_grp(off):
                for t in range(16):
                    def proc(off, t=t):
                        w = g * 16 + t
                        for u in range(8):
                            s = row_v[pl.ds(w * 128 + u * 16, 16)]
                            m = s >= lb_f
                            pos = jnp.minimum(off + _iota16(),
                                              jnp.int32(CAP - 1))
                            ivec = _iota16() + (w * 128 + u * 16)
                            plsc.store_scatter(svalf_v, [pos],
                                               jnp.where(m, s, neg))
                            plsc.store_scatter(sidx_v, [pos], ivec)
                            pc = plsc.all_reduce_population_count(m)
                            off = off + jnp.where(
                                pc > 0, jnp.int32(16), jnp.int32(0))
                        return off

                    off = lax.cond(
                        ((bits >> jnp.int32(t)) & jnp.int32(1))
                        != jnp.int32(0),
                        proc, lambda o: o, off)
                return off

            return lax.cond(bits != jnp.int32(0), proc_grp,
                            lambda o: o, off)

        off = lax.fori_loop(0, NWIN // 16, grp_body,
                            jnp.zeros((16,), jnp.int32))
        n_surv = jnp.max(off)
        overflow = n_surv > jnp.int32(CAP)
        nv_used = jnp.minimum((n_surv + 15) // 16, jnp.int32(NVCAP))

        if _EXP >= 2:
            return carry

        def map_sv(i, c):
            svalu_v[pl.ds(i * 16, 16)] = _to_u32(svalf_v[pl.ds(i * 16, 16)])
            return c
        lax.fori_loop(0, nv_used, map_sv, jnp.int32(0))

        def read_sv(i):
            return svalu_v[pl.ds(i * 16, 16)]

        def read_row_u(i):
            return _to_u32(row_v[pl.ds(i * 16, 16)])

        v96_u, q_eq = lax.cond(
            overflow,
            lambda: _select_kth(read_row_u, NVROW, K)(hist_v),
            lambda: _select_kth(read_sv, nv_used, K)(hist_v))
        v96_vec = jnp.full((16,), v96_u)

        def emit(read_u, read_idx, nv):
            def body(i, carry):
                nout, eq_seen = carry
                u = read_u(i)
                m_gt = u > v96_vec
                m_eq = u == v96_vec
                eqc = plsc.cumsum(m_eq.astype(jnp.int32))
                take_eq = m_eq & ((eq_seen + eqc) <= q_eq)
                m = m_gt | take_eq
                mi = m.astype(jnp.int32)
                pos = jnp.minimum(nout + plsc.cumsum(mi) - 1,
                                  jnp.int32(K - 1))
                plsc.store_scatter(fidx_v, [pos], read_idx(i), mask=m)
                nout = nout + plsc.all_reduce_population_count(m)
                eq_seen = eq_seen + plsc.all_reduce_population_count(m_eq)
                return nout, eq_seen

            return body

        zz = (jnp.zeros((16,), jnp.int32), jnp.zeros((16,), jnp.int32))

        def emit_surv(_):
            body = emit(read_sv, lambda i: sidx_v[pl.ds(i * 16, 16)], NVCAP)
            lax.fori_loop(0, nv_used, body, zz)
            return jnp.int32(0)

        def emit_full(_):
            body = emit(read_row_u, lambda i: _iota16() + i * 16, NVROW)
            lax.fori_loop(0, NVROW, body, zz)
            return jnp.int32(0)

        lax.cond(overflow, emit_full, emit_surv, jnp.int32(0))

        if _EXP < 1:
            pltpu.async_copy(ck.at[fidx_v], ckrows_v, sem).wait()
            pltpu.sync_copy(ckrows_v, ctxk_out.at[row])
        return carry

    lax.fori_loop(0, ROWS_PER_W, do_row, jnp.int32(0))


def _stage_b(scores, wmax, ck):
    mesh = plsc.VectorSubcoreMesh(core_axis_name="c", subcore_axis_name="s")
    f = pl.kernel(
        _sc_body,
        mesh=mesh,
        compiler_params=pltpu.CompilerParams(needs_layout_passes=False),
        out_type=jax.ShapeDtypeStruct((B, K, 128), jnp.float32),
        scratch_types=[
            pltpu.VMEM((NPAD,), jnp.float32),
            pltpu.VMEM((NWIN,), jnp.float32),
            pltpu.VMEM((NWIN,), jnp.uint32),
            pltpu.VMEM((16,), jnp.int32),
            pltpu.VMEM((CAP,), jnp.float32),
            pltpu.VMEM((CAP,), jnp.uint32),
            pltpu.VMEM((CAP,), jnp.int32),
            pltpu.VMEM((K,), jnp.int32),
            pltpu.VMEM((K, 128), jnp.float32),
            pltpu.SemaphoreType.DMA,
        ],
    )
    return f(scores, wmax, ck)


BBLK = 64
NCPAD = 128


def _final_body(xe_ref, xk_ref, ctxk_ref, le_ref,
                w1_ref, b1_ref, w2_ref, out_ref):
    xe = xe_ref[...]
    xk = xk_ref[...]
    ctxk = ctxk_ref[..., :H]
    labels = ctxk_ref[..., 127].astype(jnp.int32)
    diff3 = xk[:, None, :] - ctxk
    d2 = jnp.sum(diff3 * diff3, axis=2)
    tv = -jnp.sqrt(jnp.maximum(d2, 1e-12))
    m = jnp.max(tv, axis=1, keepdims=True)
    e = jnp.exp(tv - m)
    attn = e / jnp.sum(e, axis=1, keepdims=True)

    diff = diff3.reshape(BBLK * K, H)
    h = jnp.dot(diff, w1_ref[...].T, preferred_element_type=jnp.float32)
    h = jnp.maximum(h + b1_ref[...], 0.0)
    h = jnp.dot(h, w2_ref[...].T, preferred_element_type=jnp.float32)

    iota_c = lax.broadcasted_iota(jnp.int32, (BBLK, K, NCPAD), 2)
    onehot = (labels[:, :, None] == iota_c).astype(
        jnp.float32).reshape(BBLK * K, NCPAD)
    labv = jnp.dot(onehot, le_ref[...], preferred_element_type=jnp.float32)

    tot = (labv + h).reshape(BBLK, K, H)
    ctx = jnp.sum(attn[:, :, None] * tot, axis=1)
    out_ref[...] = xe + ctx


def _stage_c(xe, xk, ctxk, label_emb_pad, t_w1, t_b1, t_w2):
    return pl.pallas_call(
        _final_body,
        grid=(B // BBLK,),
        in_specs=[
            pl.BlockSpec((BBLK, H), lambda i: (i, 0)),
            pl.BlockSpec((BBLK, H), lambda i: (i, 0)),
            pl.BlockSpec((BBLK, K, 128), lambda i: (i, 0, 0)),
            pl.BlockSpec((NCPAD, H), lambda i: (0, 0)),
            pl.BlockSpec((H, H), lambda i: (0, 0)),
            pl.BlockSpec((1, H), lambda i: (0, 0)),
            pl.BlockSpec((H, H), lambda i: (0, 0)),
        ],
        out_specs=pl.BlockSpec((BBLK, H), lambda i: (i, 0)),
        out_shape=jax.ShapeDtypeStruct((B, H), jnp.float32),
    )(xe, xk, ctxk, label_emb_pad, t_w1, t_b1.reshape(1, H), t_w2)


def kernel(x, candidate_x, candidate_labels, enc_w, enc_b, key_w, key_b,
           val_w, val_b, label_emb, t_w1, t_b1, t_w2):
    del val_w, val_b
    labf = jnp.pad(candidate_labels.astype(jnp.float32), (0, NPAD - N))
    labf = labf.reshape(NPAD, 1)
    cx_pad = jnp.pad(candidate_x, ((0, NPAD - N), (0, 0)))
    le_pad = jnp.pad(label_emb, ((0, NCPAD - NC), (0, 0)))

    xe, xk = _encode_queries(x, enc_w, enc_b, key_w, key_b)
    ck, scores, wmax3 = _stage_a(cx_pad, labf, enc_w, enc_b, key_w,
                                 key_b, xk)
    wmax = jnp.transpose(wmax3, (1, 0, 2)).reshape(B, NWIN)

    ctxk = _stage_b(scores, wmax, ck)

    return _stage_c(xe, xk, ctxk, le_pad, t_w1, t_b1, t_w2)


# speedup vs baseline: 17.2005x; 1.0002x over previous
import functools

import jax
import jax.numpy as jnp
from jax import lax
from jax.experimental import pallas as pl
from jax.experimental.pallas import tpu as pltpu
from jax.experimental.pallas import tpu_sc as plsc

B = 512
N = 100000
D = 64
H = 64
K = 96
NC = 100
CHUNK = 2048
NPAD = 100352
NSTEPS = NPAD // CHUNK
WIN = 128
NWIN = NPAD // WIN
NEG = -3.0e38


def _enc_body(x_ref, ew_ref, eb_ref, kw_ref, kb_ref, xe_ref, xk_ref):
    xe = jnp.dot(x_ref[...], ew_ref[...].T,
                 preferred_element_type=jnp.float32) + eb_ref[...]
    xe_ref[...] = xe
    xk_ref[...] = jnp.dot(xe, kw_ref[...].T,
                          preferred_element_type=jnp.float32) + kb_ref[...]


def _encode_queries(x, enc_w, enc_b, key_w, key_b):
    return pl.pallas_call(
        _enc_body,
        out_shape=(jax.ShapeDtypeStruct((B, H), jnp.float32),
                   jax.ShapeDtypeStruct((B, H), jnp.float32)),
    )(x, enc_w, enc_b.reshape(1, H), key_w, key_b.reshape(1, H))


def _scores_body(cx_ref, lab_ref, ew_ref, eb_ref, kw_ref, kb_ref, xk_ref,
                 ck_ref, sc_ref, wm_ref):
    i = pl.program_id(0)
    ce = jnp.dot(cx_ref[...], ew_ref[...].T,
                 preferred_element_type=jnp.float32) + eb_ref[...]
    ck = jnp.dot(ce, kw_ref[...].T,
                 preferred_element_type=jnp.float32) + kb_ref[...]
    ck_ref[...] = jnp.concatenate(
        [ck, jnp.zeros((CHUNK, 63), jnp.float32), lab_ref[...]], axis=1)
    cn2 = jnp.sum(ck * ck, axis=1)
    xc = lax.dot_general(xk_ref[...], ck,
                         (((1,), (1,)), ((), ())),
                         preferred_element_type=jnp.float32)
    col = i * CHUNK + lax.broadcasted_iota(jnp.int32, (1, CHUNK), 1)
    sc = jnp.where(col < N, xc - 0.5 * cn2[None, :], NEG)
    sc_ref[...] = sc
    wm_ref[...] = jnp.max(sc.reshape(B, CHUNK // WIN, WIN), axis=2)[None]


def _stage_a(cx_pad, labf, enc_w, enc_b, key_w, key_b, xk):
    return pl.pallas_call(
        _scores_body,
        grid=(NSTEPS,),
        in_specs=[
            pl.BlockSpec((CHUNK, D), lambda i: (i, 0)),
            pl.BlockSpec((CHUNK, 1), lambda i: (i, 0)),
            pl.BlockSpec((H, D), lambda i: (0, 0)),
            pl.BlockSpec((1, H), lambda i: (0, 0)),
            pl.BlockSpec((H, H), lambda i: (0, 0)),
            pl.BlockSpec((1, H), lambda i: (0, 0)),
            pl.BlockSpec((B, H), lambda i: (0, 0)),
        ],
        out_specs=(
            pl.BlockSpec((CHUNK, 128), lambda i: (i, 0)),
            pl.BlockSpec((B, CHUNK), lambda i: (0, i)),
            pl.BlockSpec((1, B, CHUNK // WIN), lambda i: (i, 0, 0)),
        ),
        out_shape=(
            jax.ShapeDtypeStruct((NPAD, 128), jnp.float32),
            jax.ShapeDtypeStruct((B, NPAD), jnp.float32),
            jax.ShapeDtypeStruct((NSTEPS, B, CHUNK // WIN), jnp.float32),
        ),
    )(cx_pad, labf, enc_w, enc_b.reshape(1, H), key_w,
      key_b.reshape(1, H), xk)


CAP = 4096
NVROW = NPAD // 16
NVCAP = CAP // 16
NVWIN = NWIN // 16
ROWS_PER_W = B // 32

def _to_u32(f):
    ub = lax.bitcast_convert_type(f, jnp.uint32)
    neg = (ub >> jnp.uint32(31)) == jnp.uint32(1)
    return jnp.where(neg, ~ub, ub | jnp.uint32(0x80000000))


def _iota16():
    return lax.broadcasted_iota(jnp.int32, (16,), 0)


def _select_kth(read_u, nv, k, rounds=8):
    prefix = jnp.uint32(0)
    k_rem = jnp.int32(k)
    ones = jnp.ones((16,), jnp.int32)

    def hist_round(shift, prefix, k_rem, first, hist_ref):
        hist_ref[...] = jnp.zeros((16,), jnp.int32)
        sh = jnp.uint32(shift)

        def body(i, carry):
            u = read_u(i)
            if first:
                m = jnp.ones((16,), jnp.bool_)
            else:
                m = (u >> jnp.uint32(shift + 4)) == (
                    prefix >> jnp.uint32(shift + 4))
            digit = ((u >> sh) & jnp.uint32(15)).astype(jnp.int32)
            plsc.addupdate_scatter(hist_ref, [digit], ones, mask=m)
            return carry

        lax.fori_loop(0, nv, body, jnp.int32(0))
        h = hist_ref[...]
        rh = lax.rev(h, (0,))
        c = plsc.cumsum(rh)
        ge = c >= k_rem
        i_star = jnp.max(plsc.all_reduce_ffs(ge))
        cnt_gt = jnp.sum(jnp.where(_iota16() < i_star, rh, 0))
        d = (jnp.int32(15) - i_star).astype(jnp.uint32)
        prefix = prefix | (d << sh)
        k_rem = k_rem - cnt_gt
        return prefix, k_rem

    def run(hist_ref):
        p, kr = prefix, k_rem
        for r in range(rounds):
            p, kr = hist_round(28 - 4 * r, p, kr, r == 0, hist_ref)
        return p, kr

    return run


_EXP = 0


def _sc_body(scores, wmax, ck, ctxk_out,
             row_v, wmf_v, wmu_v, hist_v, pos_v, svalf_v, svalu_v, sidx_v,
             fidx_v, ckrows_v, sem):
    wid = lax.axis_index("s") * 2 + lax.axis_index("c")
    neg = jnp.full((16,), NEG, jnp.float32)

    def do_row(j, carry):
        row = wid * ROWS_PER_W + j
        row_cp = pltpu.async_copy(scores.at[row], row_v, sem)
        pltpu.sync_copy(wmax.at[row], wmf_v)

        def map_wm(i, c):
            wmu_v[pl.ds(i * 16, 16)] = _to_u32(wmf_v[pl.ds(i * 16, 16)])
            return c
        lax.fori_loop(0, NVWIN, map_wm, jnp.int32(0))

        def read_wm(i):
            return wmu_v[pl.ds(i * 16, 16)]
        lb_u, _ = _select_kth(read_wm, NVWIN, K, rounds=4)(hist_v)
        lb_uv = jnp.full((16,), lb_u)
        lb_f = jnp.min(lax.bitcast_convert_type(
            jnp.where((lb_uv >> jnp.uint32(31)) == jnp.uint32(1),
                      lb_uv & jnp.uint32(0x7FFFFFFF),
                      ~lb_uv),
            jnp.float32))

        row_cp.wait()
        if _EXP >= 3:
            return carry

        def grp_body(g, off):
            wmv = wmf_v[pl.ds(g * 16, 16)]
            hit = wmv >= lb_f
            bits = jnp.sum(jnp.where(hit, jnp.int32(1) << _iota16(),
                                     jnp.int32(0)))
            if _EXP == 5:
                bits = bits & jnp.int32(0)

            def proc_grp(off):
                for t in range(16):
                    def proc(off, t=t):
                        w = g * 16 + t
                        for u in range(8):
                            s = row_v[pl.ds(w * 128 + u * 16, 16)]
                            m = s >= lb_f
                            pos = jnp.minimum(off + _iota16(),
                                              jnp.int32(CAP - 1))
                            ivec = _iota16() + (w * 128 + u * 16)
                            plsc.store_scatter(svalf_v, [pos],
                                               jnp.where(m, s, neg))
                            plsc.store_scatter(sidx_v, [pos], ivec)
                            pc = plsc.all_reduce_population_count(m)
                            off = off + jnp.where(
                                pc > 0, jnp.int32(16), jnp.int32(0))
                        return off

                    off = lax.cond(
                        ((bits >> jnp.int32(t)) & jnp.int32(1))
                        != jnp.int32(0),
                        proc, lambda o: o, off)
                return off

            return lax.cond(bits != jnp.int32(0), proc_grp,
                            lambda o: o, off)

        off = lax.fori_loop(0, NWIN // 16, grp_body,
                            jnp.zeros((16,), jnp.int32))
        n_surv = jnp.max(off)
        overflow = n_surv > jnp.int32(CAP)
        nv_used = jnp.minimum((n_surv + 15) // 16, jnp.int32(NVCAP))

        if _EXP >= 2:
            return carry

        def map_sv(i, c):
            svalu_v[pl.ds(i * 16, 16)] = _to_u32(svalf_v[pl.ds(i * 16, 16)])
            return c
        lax.fori_loop(0, nv_used, map_sv, jnp.int32(0))

        def read_sv(i):
            return svalu_v[pl.ds(i * 16, 16)]

        def read_row_u(i):
            return _to_u32(row_v[pl.ds(i * 16, 16)])

        v96_u, q_eq = lax.cond(
            overflow,
            lambda: _select_kth(read_row_u, NVROW, K)(hist_v),
            lambda: _select_kth(read_sv, nv_used, K)(hist_v))
        v96_vec = jnp.full((16,), v96_u)

        def emit(read_u, read_idx, nv):
            def body(i, carry):
                nout, eq_seen = carry
                u = read_u(i)
                m_gt = u > v96_vec
                m_eq = u == v96_vec
                eqc0 = plsc.cumsum(m_eq.astype(jnp.int32))
                pos_v[pl.ds(16, 16)] = eqc0
                eqc = pos_v[pl.ds(16, 16)]
                take_eq = m_eq & ((eq_seen + eqc) <= q_eq)
                m = m_gt | take_eq
                mi = m.astype(jnp.int32)
                pos_v[pl.ds(0, 16)] = plsc.cumsum(mi)
                pcv = pos_v[pl.ds(0, 16)]
                pos = jnp.minimum(nout + pcv - 1, jnp.int32(K - 1))
                plsc.store_scatter(fidx_v, [pos], read_idx(i), mask=m)
                nout = nout + plsc.all_reduce_population_count(m)
                eq_seen = eq_seen + plsc.all_reduce_population_count(m_eq)
                return nout, eq_seen

            return body

        zz = (jnp.zeros((16,), jnp.int32), jnp.zeros((16,), jnp.int32))

        def emit_surv(_):
            body = emit(read_sv, lambda i: sidx_v[pl.ds(i * 16, 16)], NVCAP)
            lax.fori_loop(0, nv_used, body, zz)
            return jnp.int32(0)

        def emit_full(_):
            body = emit(read_row_u, lambda i: _iota16() + i * 16, NVROW)
            lax.fori_loop(0, NVROW, body, zz)
            return jnp.int32(0)

        lax.cond(overflow, emit_full, emit_surv, jnp.int32(0))

        if _EXP < 1:
            pltpu.async_copy(ck.at[fidx_v], ckrows_v, sem).wait()
            pltpu.sync_copy(ckrows_v, ctxk_out.at[row])
        return carry

    lax.fori_loop(0, ROWS_PER_W, do_row, jnp.int32(0))


def _stage_b(scores, wmax, ck):
    mesh = plsc.VectorSubcoreMesh(core_axis_name="c", subcore_axis_name="s")
    f = pl.kernel(
        _sc_body,
        mesh=mesh,
        compiler_params=pltpu.CompilerParams(needs_layout_passes=False),
        out_type=jax.ShapeDtypeStruct((B, K, 128), jnp.float32),
        scratch_types=[
            pltpu.VMEM((NPAD,), jnp.float32),
            pltpu.VMEM((NWIN,), jnp.float32),
            pltpu.VMEM((NWIN,), jnp.uint32),
            pltpu.VMEM((16,), jnp.int32),
            pltpu.VMEM((32,), jnp.int32),
            pltpu.VMEM((CAP,), jnp.float32),
            pltpu.VMEM((CAP,), jnp.uint32),
            pltpu.VMEM((CAP,), jnp.int32),
            pltpu.VMEM((K,), jnp.int32),
            pltpu.VMEM((K, 128), jnp.float32),
            pltpu.SemaphoreType.DMA,
        ],
    )
    return f(scores, wmax, ck)


BBLK = 64
NCPAD = 128


def _final_body(xe_ref, xk_ref, ctxk_ref, le_ref,
                w1_ref, b1_ref, w2_ref, out_ref):
    xe = xe_ref[...]
    xk = xk_ref[...]
    ctxk = ctxk_ref[..., :H]
    labels = ctxk_ref[..., 127].astype(jnp.int32)
    diff3 = xk[:, None, :] - ctxk
    d2 = jnp.sum(diff3 * diff3, axis=2)
    tv = -jnp.sqrt(jnp.maximum(d2, 1e-12))
    m = jnp.max(tv, axis=1, keepdims=True)
    e = jnp.exp(tv - m)
    attn = e / jnp.sum(e, axis=1, keepdims=True)

    diff = diff3.reshape(BBLK * K, H)
    h = jnp.dot(diff, w1_ref[...].T, preferred_element_type=jnp.float32)
    h = jnp.maximum(h + b1_ref[...], 0.0)
    h = jnp.dot(h, w2_ref[...].T, preferred_element_type=jnp.float32)

    iota_c = lax.broadcasted_iota(jnp.int32, (BBLK, K, NCPAD), 2)
    onehot = (labels[:, :, None] == iota_c).astype(
        jnp.float32).reshape(BBLK * K, NCPAD)
    labv = jnp.dot(onehot, le_ref[...], preferred_element_type=jnp.float32)

    tot = (labv + h).reshape(BBLK, K, H)
    ctx = jnp.sum(attn[:, :, None] * tot, axis=1)
    out_ref[...] = xe + ctx


def _stage_c(xe, xk, ctxk, label_emb_pad, t_w1, t_b1, t_w2):
    return pl.pallas_call(
        _final_body,
        grid=(B // BBLK,),
        in_specs=[
            pl.BlockSpec((BBLK, H), lambda i: (i, 0)),
            pl.BlockSpec((BBLK, H), lambda i: (i, 0)),
            pl.BlockSpec((BBLK, K, 128), lambda i: (i, 0, 0)),
            pl.BlockSpec((NCPAD, H), lambda i: (0, 0)),
            pl.BlockSpec((H, H), lambda i: (0, 0)),
            pl.BlockSpec((1, H), lambda i: (0, 0)),
            pl.BlockSpec((H, H), lambda i: (0, 0)),
        ],
        out_specs=pl.BlockSpec((BBLK, H), lambda i: (i, 0)),
        out_shape=jax.ShapeDtypeStruct((B, H), jnp.float32),
    )(xe, xk, ctxk, label_emb_pad, t_w1, t_b1.reshape(1, H), t_w2)


def kernel(x, candidate_x, candidate_labels, enc_w, enc_b, key_w, key_b,
           val_w, val_b, label_emb, t_w1, t_b1, t_w2):
    del val_w, val_b
    labf = jnp.pad(candidate_labels.astype(jnp.float32), (0, NPAD - N))
    labf = labf.reshape(NPAD, 1)
    cx_pad = jnp.pad(candidate_x, ((0, NPAD - N), (0, 0)))
    le_pad = jnp.pad(label_emb, ((0, NCPAD - NC), (0, 0)))

    xe, xk = _encode_queries(x, enc_w, enc_b, key_w, key_b)
    ck, scores, wmax3 = _stage_a(cx_pad, labf, enc_w, enc_b, key_w,
                                 key_b, xk)
    wmax = jnp.transpose(wmax3, (1, 0, 2)).reshape(B, NWIN)

    ctxk = _stage_b(scores, wmax, ck)

    return _stage_c(xe, xk, ctxk, le_pad, t_w1, t_b1, t_w2)


# speedup vs baseline: 20.7543x; 1.2066x over previous
import functools

import jax
import jax.numpy as jnp
from jax import lax
from jax.experimental import pallas as pl
from jax.experimental.pallas import tpu as pltpu
from jax.experimental.pallas import tpu_sc as plsc

B = 512
N = 100000
D = 64
H = 64
K = 96
NC = 100
CHUNK = 2048
NPAD = 100352
NSTEPS = NPAD // CHUNK
WIN = 128
NWIN = NPAD // WIN
NEG = -3.0e38


def _enc_body(x_ref, ew_ref, eb_ref, kw_ref, kb_ref, xe_ref, xk_ref):
    xe = jnp.dot(x_ref[...], ew_ref[...].T,
                 preferred_element_type=jnp.float32) + eb_ref[...]
    xe_ref[...] = xe
    xk_ref[...] = jnp.dot(xe, kw_ref[...].T,
                          preferred_element_type=jnp.float32) + kb_ref[...]


def _encode_queries(x, enc_w, enc_b, key_w, key_b):
    return pl.pallas_call(
        _enc_body,
        out_shape=(jax.ShapeDtypeStruct((B, H), jnp.float32),
                   jax.ShapeDtypeStruct((B, H), jnp.float32)),
    )(x, enc_w, enc_b.reshape(1, H), key_w, key_b.reshape(1, H))


def _scores_body(cx_ref, lab_ref, ew_ref, eb_ref, kw_ref, kb_ref, xk_ref,
                 ck_ref, sc_ref, wm_ref):
    i = pl.program_id(0)
    ce = jnp.dot(cx_ref[...], ew_ref[...].T,
                 preferred_element_type=jnp.float32) + eb_ref[...]
    ck = jnp.dot(ce, kw_ref[...].T,
                 preferred_element_type=jnp.float32) + kb_ref[...]
    ck_ref[...] = jnp.concatenate(
        [ck, jnp.zeros((CHUNK, 63), jnp.float32), lab_ref[...]], axis=1)
    cn2 = jnp.sum(ck * ck, axis=1)
    xc = lax.dot_general(xk_ref[...], ck,
                         (((1,), (1,)), ((), ())),
                         preferred_element_type=jnp.float32)
    col = i * CHUNK + lax.broadcasted_iota(jnp.int32, (1, CHUNK), 1)
    sc = jnp.where(col < N, xc - 0.5 * cn2[None, :], NEG)
    sc_ref[...] = sc
    wm_ref[...] = jnp.max(sc.reshape(B, CHUNK // WIN, WIN), axis=2)[None]


def _stage_a(cx_pad, labf, enc_w, enc_b, key_w, key_b, xk):
    return pl.pallas_call(
        _scores_body,
        grid=(NSTEPS,),
        in_specs=[
            pl.BlockSpec((CHUNK, D), lambda i: (i, 0)),
            pl.BlockSpec((CHUNK, 1), lambda i: (i, 0)),
            pl.BlockSpec((H, D), lambda i: (0, 0)),
            pl.BlockSpec((1, H), lambda i: (0, 0)),
            pl.BlockSpec((H, H), lambda i: (0, 0)),
            pl.BlockSpec((1, H), lambda i: (0, 0)),
            pl.BlockSpec((B, H), lambda i: (0, 0)),
        ],
        out_specs=(
            pl.BlockSpec((CHUNK, 128), lambda i: (i, 0)),
            pl.BlockSpec((B, CHUNK), lambda i: (0, i)),
            pl.BlockSpec((1, B, CHUNK // WIN), lambda i: (i, 0, 0)),
        ),
        out_shape=(
            jax.ShapeDtypeStruct((NPAD, 128), jnp.float32),
            jax.ShapeDtypeStruct((B, NPAD), jnp.float32),
            jax.ShapeDtypeStruct((NSTEPS, B, CHUNK // WIN), jnp.float32),
        ),
    )(cx_pad, labf, enc_w, enc_b.reshape(1, H), key_w,
      key_b.reshape(1, H), xk)


CAP = 4096
NVROW = NPAD // 16
NVCAP = CAP // 16
NVWIN = NWIN // 16
ROWS_PER_W = B // 32

def _to_u32(f):
    ub = lax.bitcast_convert_type(f, jnp.uint32)
    neg = (ub >> jnp.uint32(31)) == jnp.uint32(1)
    return jnp.where(neg, ~ub, ub | jnp.uint32(0x80000000))


def _iota16():
    return lax.broadcasted_iota(jnp.int32, (16,), 0)


def _select_kth(read_u, nv, k, rounds=8):
    prefix = jnp.uint32(0)
    k_rem = jnp.int32(k)
    ones = jnp.ones((16,), jnp.int32)

    def hist_round(shift, prefix, k_rem, first, hist_ref):
        hist_ref[...] = jnp.zeros((16,), jnp.int32)
        sh = jnp.uint32(shift)

        def body(i, carry):
            u = read_u(i)
            if first:
                m = jnp.ones((16,), jnp.bool_)
            else:
                m = (u >> jnp.uint32(shift + 4)) == (
                    prefix >> jnp.uint32(shift + 4))
            digit = ((u >> sh) & jnp.uint32(15)).astype(jnp.int32)
            plsc.addupdate_scatter(hist_ref, [digit], ones, mask=m)
            return carry

        lax.fori_loop(0, nv, body, jnp.int32(0))
        h = hist_ref[...]
        rh = lax.rev(h, (0,))
        c = plsc.cumsum(rh)
        ge = c >= k_rem
        i_star = jnp.max(plsc.all_reduce_ffs(ge))
        cnt_gt = jnp.sum(jnp.where(_iota16() < i_star, rh, 0))
        d = (jnp.int32(15) - i_star).astype(jnp.uint32)
        prefix = prefix | (d << sh)
        k_rem = k_rem - cnt_gt
        return prefix, k_rem

    def run(hist_ref):
        p, kr = prefix, k_rem
        for r in range(rounds):
            p, kr = hist_round(28 - 4 * r, p, kr, r == 0, hist_ref)
        return p, kr

    return run


_EXP = 0


def _sc_body(scores, wmax, ck, ctxk_out,
             row_v, wmf_v, wmu_v, hist_v, pos_v, svalf_v, svalu_v, sidx_v,
             fidx_v, ckrows_v, sem):
    wid = lax.axis_index("s") * 2 + lax.axis_index("c")
    neg = jnp.full((16,), NEG, jnp.float32)

    def do_row(j, carry):
        row = wid * ROWS_PER_W + j
        row_cp = pltpu.async_copy(scores.at[row], row_v, sem)
        pltpu.sync_copy(wmax.at[row], wmf_v)

        def map_wm(i, c):
            wmu_v[pl.ds(i * 16, 16)] = _to_u32(wmf_v[pl.ds(i * 16, 16)])
            return c
        lax.fori_loop(0, NVWIN, map_wm, jnp.int32(0))

        def read_wm(i):
            return wmu_v[pl.ds(i * 16, 16)]
        lb_u, _ = _select_kth(read_wm, NVWIN, K, rounds=4)(hist_v)
        lb_uv = jnp.full((16,), lb_u)
        lb_f = jnp.min(lax.bitcast_convert_type(
            jnp.where((lb_uv >> jnp.uint32(31)) == jnp.uint32(1),
                      lb_uv & jnp.uint32(0x7FFFFFFF),
                      ~lb_uv),
            jnp.float32))

        row_cp.wait()
        if _EXP >= 3:
            return carry

        def grp_body(g, off):
            wmv = wmf_v[pl.ds(g * 16, 16)]
            hit = wmv >= lb_f
            bits = jnp.sum(jnp.where(hit, jnp.int32(1) << _iota16(),
                                     jnp.int32(0)))
            if _EXP == 5:
                bits = bits & jnp.int32(0)

            def proc_grp(off):
                for t in range(16):
                    def proc(off, t=t):
                        w = g * 16 + t
                        for u in range(8):
                            s = row_v[pl.ds(w * 128 + u * 16, 16)]
                            m = s >= lb_f
                            pos = jnp.minimum(off + _iota16(),
                                              jnp.int32(CAP - 1))
                            ivec = _iota16() + (w * 128 + u * 16)
                            plsc.store_scatter(svalf_v, [pos],
                                               jnp.where(m, s, neg))
                            plsc.store_scatter(sidx_v, [pos], ivec)
                            pc = plsc.all_reduce_population_count(m)
                            off = off + jnp.where(
                                pc > 0, jnp.int32(16), jnp.int32(0))
                        return off

                    off = lax.cond(
                        ((bits >> jnp.int32(t)) & jnp.int32(1))
                        != jnp.int32(0),
                        proc, lambda o: o, off)
                return off

            return lax.cond(bits != jnp.int32(0), proc_grp,
                            lambda o: o, off)

        off = lax.fori_loop(0, NWIN // 16, grp_body,
                            jnp.zeros((16,), jnp.int32))
        n_surv = jnp.max(off)
        overflow = n_surv > jnp.int32(CAP)
        nv_used = jnp.minimum((n_surv + 15) // 16, jnp.int32(NVCAP))

        if _EXP >= 2:
            return carry

        neg_thr = jnp.float32(-2.0e38)

        def compact(i, coff):
            s = svalf_v[pl.ds(i * 16, 16)]
            ix = sidx_v[pl.ds(i * 16, 16)]
            m = s > neg_thr
            pos_v[pl.ds(0, 16)] = plsc.cumsum(m.astype(jnp.int32))
            pcv = pos_v[pl.ds(0, 16)]
            pos = jnp.minimum(coff + pcv - 1, jnp.int32(CAP - 1))
            plsc.store_scatter(svalf_v, [pos], s, mask=m)
            plsc.store_scatter(sidx_v, [pos], ix, mask=m)
            return coff + plsc.all_reduce_population_count(m)

        coff = lax.fori_loop(0, nv_used, compact,
                             jnp.zeros((16,), jnp.int32))
        n_real = jnp.max(coff)
        nv2 = jnp.minimum((n_real + 15) // 16, jnp.int32(NVCAP))
        tpos = jnp.minimum(n_real + _iota16(), jnp.int32(CAP - 1))
        plsc.store_scatter(svalf_v, [tpos], neg,
                           mask=_iota16() < (nv2 * 16 - n_real))

        def map_sv(i, c):
            svalu_v[pl.ds(i * 16, 16)] = _to_u32(svalf_v[pl.ds(i * 16, 16)])
            return c
        lax.fori_loop(0, nv2, map_sv, jnp.int32(0))

        def read_sv(i):
            return svalu_v[pl.ds(i * 16, 16)]

        def read_row_u(i):
            return _to_u32(row_v[pl.ds(i * 16, 16)])

        v96_u, q_eq = lax.cond(
            overflow,
            lambda: _select_kth(read_row_u, NVROW, K)(hist_v),
            lambda: _select_kth(read_sv, nv2, K)(hist_v))
        v96_vec = jnp.full((16,), v96_u)

        def emit(read_u, read_idx, nv):
            def body(i, carry):
                nout, eq_seen = carry
                u = read_u(i)
                m_gt = u > v96_vec
                m_eq = u == v96_vec
                eqc0 = plsc.cumsum(m_eq.astype(jnp.int32))
                pos_v[pl.ds(16, 16)] = eqc0
                eqc = pos_v[pl.ds(16, 16)]
                take_eq = m_eq & ((eq_seen + eqc) <= q_eq)
                m = m_gt | take_eq
                mi = m.astype(jnp.int32)
                pos_v[pl.ds(0, 16)] = plsc.cumsum(mi)
                pcv = pos_v[pl.ds(0, 16)]
                pos = jnp.minimum(nout + pcv - 1, jnp.int32(K - 1))
                plsc.store_scatter(fidx_v, [pos], read_idx(i), mask=m)
                nout = nout + plsc.all_reduce_population_count(m)
                eq_seen = eq_seen + plsc.all_reduce_population_count(m_eq)
                return nout, eq_seen

            return body

        zz = (jnp.zeros((16,), jnp.int32), jnp.zeros((16,), jnp.int32))

        def emit_surv(_):
            body = emit(read_sv, lambda i: sidx_v[pl.ds(i * 16, 16)], NVCAP)
            lax.fori_loop(0, nv2, body, zz)
            return jnp.int32(0)

        def emit_full(_):
            body = emit(read_row_u, lambda i: _iota16() + i * 16, NVROW)
            lax.fori_loop(0, NVROW, body, zz)
            return jnp.int32(0)

        lax.cond(overflow, emit_full, emit_surv, jnp.int32(0))

        if _EXP < 1:
            pltpu.async_copy(ck.at[fidx_v], ckrows_v, sem).wait()
            pltpu.sync_copy(ckrows_v, ctxk_out.at[row])
        return carry

    lax.fori_loop(0, ROWS_PER_W, do_row, jnp.int32(0))


def _stage_b(scores, wmax, ck):
    mesh = plsc.VectorSubcoreMesh(core_axis_name="c", subcore_axis_name="s")
    f = pl.kernel(
        _sc_body,
        mesh=mesh,
        compiler_params=pltpu.CompilerParams(needs_layout_passes=False),
        out_type=jax.ShapeDtypeStruct((B, K, 128), jnp.float32),
        scratch_types=[
            pltpu.VMEM((NPAD,), jnp.float32),
            pltpu.VMEM((NWIN,), jnp.float32),
            pltpu.VMEM((NWIN,), jnp.uint32),
            pltpu.VMEM((16,), jnp.int32),
            pltpu.VMEM((32,), jnp.int32),
            pltpu.VMEM((CAP,), jnp.float32),
            pltpu.VMEM((CAP,), jnp.uint32),
            pltpu.VMEM((CAP,), jnp.int32),
            pltpu.VMEM((K,), jnp.int32),
            pltpu.VMEM((K, 128), jnp.float32),
            pltpu.SemaphoreType.DMA,
        ],
    )
    return f(scores, wmax, ck)


BBLK = 64
NCPAD = 128


def _final_body(xe_ref, xk_ref, ctxk_ref, le_ref,
                w1_ref, b1_ref, w2_ref, out_ref):
    xe = xe_ref[...]
    xk = xk_ref[...]
    ctxk = ctxk_ref[..., :H]
    labels = ctxk_ref[..., 127].astype(jnp.int32)
    diff3 = xk[:, None, :] - ctxk
    d2 = jnp.sum(diff3 * diff3, axis=2)
    tv = -jnp.sqrt(jnp.maximum(d2, 1e-12))
    m = jnp.max(tv, axis=1, keepdims=True)
    e = jnp.exp(tv - m)
    attn = e / jnp.sum(e, axis=1, keepdims=True)

    diff = diff3.reshape(BBLK * K, H)
    h = jnp.dot(diff, w1_ref[...].T, preferred_element_type=jnp.float32)
    h = jnp.maximum(h + b1_ref[...], 0.0)
    h = jnp.dot(h, w2_ref[...].T, preferred_element_type=jnp.float32)

    iota_c = lax.broadcasted_iota(jnp.int32, (BBLK, K, NCPAD), 2)
    onehot = (labels[:, :, None] == iota_c).astype(
        jnp.float32).reshape(BBLK * K, NCPAD)
    labv = jnp.dot(onehot, le_ref[...], preferred_element_type=jnp.float32)

    tot = (labv + h).reshape(BBLK, K, H)
    ctx = jnp.sum(attn[:, :, None] * tot, axis=1)
    out_ref[...] = xe + ctx


def _stage_c(xe, xk, ctxk, label_emb_pad, t_w1, t_b1, t_w2):
    return pl.pallas_call(
        _final_body,
        grid=(B // BBLK,),
        in_specs=[
            pl.BlockSpec((BBLK, H), lambda i: (i, 0)),
            pl.BlockSpec((BBLK, H), lambda i: (i, 0)),
            pl.BlockSpec((BBLK, K, 128), lambda i: (i, 0, 0)),
            pl.BlockSpec((NCPAD, H), lambda i: (0, 0)),
            pl.BlockSpec((H, H), lambda i: (0, 0)),
            pl.BlockSpec((1, H), lambda i: (0, 0)),
            pl.BlockSpec((H, H), lambda i: (0, 0)),
        ],
        out_specs=pl.BlockSpec((BBLK, H), lambda i: (i, 0)),
        out_shape=jax.ShapeDtypeStruct((B, H), jnp.float32),
    )(xe, xk, ctxk, label_emb_pad, t_w1, t_b1.reshape(1, H), t_w2)


def kernel(x, candidate_x, candidate_labels, enc_w, enc_b, key_w, key_b,
           val_w, val_b, label_emb, t_w1, t_b1, t_w2):
    del val_w, val_b
    labf = jnp.pad(candidate_labels.astype(jnp.float32), (0, NPAD - N))
    labf = labf.reshape(NPAD, 1)
    cx_pad = jnp.pad(candidate_x, ((0, NPAD - N), (0, 0)))
    le_pad = jnp.pad(label_emb, ((0, NCPAD - NC), (0, 0)))

    xe, xk = _encode_queries(x, enc_w, enc_b, key_w, key_b)
    ck, scores, wmax3 = _stage_a(cx_pad, labf, enc_w, enc_b, key_w,
                                 key_b, xk)
    wmax = jnp.transpose(wmax3, (1, 0, 2)).reshape(B, NWIN)

    ctxk = _stage_b(scores, wmax, ck)

    return _stage_c(xe, xk, ctxk, le_pad, t_w1, t_b1, t_w2)
